# baseline (speedup 1.0000x reference)
import functools

import jax
import jax.numpy as jnp
from jax import lax
from jax.experimental import pallas as pl
from jax.experimental.pallas import tpu as pltpu
from jax.experimental.pallas import tpu_sc as plsc

N = 10000
E = 320000
D = 128
H = 128

NC = 2
NS = 16
NW = NC * NS
CHUNK = 128
NCHUNK = 80
EP = NW * NCHUNK * CHUNK
TOTCH = NW * NCHUNK
G0 = 40
G1 = 2 * NCHUNK - G0
GMAX = max(G0, G1)
NP2 = 10016
STRIPE = 624
TAIL0 = NS * STRIPE
ZR = 48
HR = 80
NPAD = HR * 128

_sc_mesh = plsc.VectorSubcoreMesh(core_axis_name="c", subcore_axis_name="s")
_sc_params = pltpu.CompilerParams(use_tc_tiling_on_sc=False,
                                  needs_layout_passes=False)


def _fill2d(ref, rows, cols, val):
    per_row = cols // 16

    def body(t, carry):
        i = t // per_row
        j = (t % per_row) * 16
        ref[i, pl.ds(j, 16)] = jnp.full((16,), val, ref.dtype)
        return carry

    lax.fori_loop(0, rows * per_row, body, 0)


def _fill1d(ref, n, val):
    def body(t, carry):
        ref[pl.ds(t * 16, 16)] = jnp.full((16,), val, ref.dtype)
        return carry

    lax.fori_loop(0, n // 16, body, 0)


@functools.partial(
    pl.kernel,
    out_type=(
        jax.ShapeDtypeStruct((NC, N, D), jnp.float32),
        jax.ShapeDtypeStruct((NW, HR, D), jnp.float32),
    ),
    mesh=_sc_mesh,
    compiler_params=_sc_params,
    scratch_types=(
        pltpu.VMEM((CHUNK,), jnp.int32),
        pltpu.VMEM((CHUNK,), jnp.int32),
        pltpu.VMEM((CHUNK,), jnp.int32),
        pltpu.VMEM((CHUNK,), jnp.int32),
        pltpu.VMEM((CHUNK, D), jnp.float32),
        pltpu.VMEM((CHUNK, D), jnp.float32),
        pltpu.VMEM((NPAD,), jnp.float32),
        pltpu.VMEM_SHARED((NP2, D), jnp.float32),
        pltpu.SemaphoreType.DMA,
        pltpu.SemaphoreType.DMA,
    ),
)
def _sc_scatter(nf, srci, dsti, sums_out, cnts_out,
                src_a, dst_a, src_b, dst_b, rows_a, rows_b,
                hist_v, sums_sh, sem_a, sem_b):
    c = lax.axis_index("c")
    s = lax.axis_index("s")
    wid = c * NS + s
    cbase = lax.select(c == 0, s * G0, NS * G0 + s * G1)
    nch = lax.select(c == 0, jnp.int32(G0), jnp.int32(G1))

    _fill2d(rows_a, CHUNK, D, 0.0)
    _fill1d(hist_v, NPAD, 0.0)

    row0 = s * STRIPE

    def zero_stripe(k, carry):
        pltpu.sync_copy(rows_a, sums_sh.at[pl.ds(row0 + k * CHUNK, CHUNK)])
        return carry

    lax.fori_loop(0, 4, zero_stripe, 0)
    pltpu.sync_copy(rows_a.at[pl.ds(0, STRIPE - 4 * CHUNK)],
                    sums_sh.at[pl.ds(row0 + 4 * CHUNK, STRIPE - 4 * CHUNK)])

    @pl.when(s == 0)
    def _zero_tail():
        pltpu.sync_copy(rows_a.at[pl.ds(0, 16)], sums_sh.at[pl.ds(TAIL0, 16)])

    plsc.subcore_barrier()

    ones16 = jnp.ones((16,), jnp.float32)

    pltpu.sync_copy(srci.at[cbase], src_a)
    pltpu.sync_copy(dsti.at[cbase], dst_a)
    pltpu.async_copy(nf.at[src_a], rows_a, sem_a)
    pltpu.sync_copy(srci.at[cbase + 1], src_b)
    pltpu.sync_copy(dsti.at[cbase + 1], dst_b)

    bufs = ((src_a, dst_a, rows_a, sem_a), (src_b, dst_b, rows_b, sem_b))

    def consume(j, cur, nxt):
        cur_s, cur_d, cur_rows, cur_sem = cur
        nxt_s, nxt_d, nxt_rows, nxt_sem = nxt

        @pl.when(j + 1 < nch)
        def _fire_next():
            pltpu.async_copy(nf.at[nxt_s], nxt_rows, nxt_sem)

        pltpu.make_async_copy(nf.at[cur_s], cur_rows, cur_sem).wait()
        pltpu.sync_copy(cur_rows, sums_sh.at[cur_d], add=True)
        for jj in range(CHUNK // 16):
            idx = cur_d[pl.ds(jj * 16, 16)]
            plsc.addupdate_scatter(hist_v, [idx], ones16)

        @pl.when(j + 2 < nch)
        def _prefetch_idx():
            pltpu.sync_copy(srci.at[cbase + j + 2], cur_s)
            pltpu.sync_copy(dsti.at[cbase + j + 2], cur_d)

    def body(k, carry):
        consume(2 * k, bufs[0], bufs[1])
        consume(2 * k + 1, bufs[1], bufs[0])
        return carry

    lax.fori_loop(0, nch // 2, body, 0)

    def repack(t, carry):
        rows_a[t // 8, pl.ds((t % 8) * 16, 16)] = hist_v[pl.ds(t * 16, 16)]
        return carry

    lax.fori_loop(0, NPAD // 16, repack, 0)
    plsc.subcore_barrier()

    pltpu.sync_copy(sums_sh.at[pl.ds(row0, STRIPE)],
                    sums_out.at[c, pl.ds(row0, STRIPE)])

    @pl.when(s == 0)
    def _dump_tail():
        pltpu.sync_copy(sums_sh.at[pl.ds(TAIL0, 16)],
                        sums_out.at[c, pl.ds(TAIL0, 16)])

    pltpu.sync_copy(rows_a.at[pl.ds(0, HR)], cnts_out.at[wid])


@functools.partial(
    pl.kernel,
    out_type=(
        jax.ShapeDtypeStruct((EP, D), jnp.float32),
        jax.ShapeDtypeStruct((EP, D), jnp.float32),
    ),
    mesh=_sc_mesh,
    compiler_params=_sc_params,
    scratch_types=(
        pltpu.VMEM((GMAX, CHUNK), jnp.int32),
        pltpu.VMEM((GMAX, CHUNK), jnp.int32),
        pltpu.VMEM((CHUNK, D), jnp.float32),
        pltpu.VMEM((CHUNK, D), jnp.float32),
        pltpu.VMEM((CHUNK, D), jnp.float32),
        pltpu.VMEM((CHUNK, D), jnp.float32),
        pltpu.SemaphoreType.DMA,
        pltpu.SemaphoreType.DMA,
        pltpu.SemaphoreType.DMA,
        pltpu.SemaphoreType.DMA,
    ),
)
def _sc_gather(table, srci, dsti, x_out, y_out,
               src_v, dst_v, x_a, y_a, x_b, y_b,
               gsem_a, gsem_b, wsem_a, wsem_b):
    c = lax.axis_index("c")
    s = lax.axis_index("s")
    cbase = lax.select(c == 0, s * G0, NS * G0 + s * G1)
    nch = lax.select(c == 0, jnp.int32(G0), jnp.int32(G1))
    base = cbase * CHUNK

    pltpu.sync_copy(srci.at[pl.ds(cbase, GMAX)], src_v)
    pltpu.sync_copy(dsti.at[pl.ds(cbase, GMAX)], dst_v)

    pltpu.async_copy(table.at[src_v.at[0]], x_a, gsem_a)
    pltpu.async_copy(table.at[dst_v.at[0]], y_a, gsem_a)

    bufs = ((x_a, y_a, gsem_a, wsem_a), (x_b, y_b, gsem_b, wsem_b))

    def consume(j, cur, nxt, first=False, last=False):
        cur_x, cur_y, cur_g, cur_w = cur
        nxt_x, nxt_y, nxt_g, nxt_w = nxt

        if not first:
            pltpu.make_async_copy(nxt_x, x_out.at[pl.ds(base, CHUNK)],
                                  nxt_w).wait()
            pltpu.make_async_copy(nxt_y, y_out.at[pl.ds(base, CHUNK)],
                                  nxt_w).wait()

        if not last:
            pltpu.async_copy(table.at[src_v.at[j + 1]], nxt_x, nxt_g)
            pltpu.async_copy(table.at[dst_v.at[j + 1]], nxt_y, nxt_g)

        pltpu.make_async_copy(table.at[src_v.at[0]], cur_x, cur_g).wait()
        pltpu.make_async_copy(table.at[dst_v.at[0]], cur_y, cur_g).wait()

        off = base + j * CHUNK
        pltpu.async_copy(cur_x, x_out.at[pl.ds(off, CHUNK)], cur_w)
        pltpu.async_copy(cur_y, y_out.at[pl.ds(off, CHUNK)], cur_w)

    consume(0, bufs[0], bufs[1], first=True)

    def body(k, carry):
        j = 2 * k + 1
        consume(j, bufs[1], bufs[0])
        consume(j + 1, bufs[0], bufs[1])
        return carry

    lax.fori_loop(0, (nch - 2) // 2, body, 0)

    consume(nch - 1, bufs[1], bufs[0], last=True)
    pltpu.make_async_copy(x_b, x_out.at[pl.ds(base, CHUNK)], wsem_b).wait()
    pltpu.make_async_copy(y_b, y_out.at[pl.ds(base, CHUNK)], wsem_b).wait()


def _mm_t(a, b):
    return lax.dot_general(a, b, (((1,), (1,)), ((), ())),
                           preferred_element_type=jnp.float32)


_BN = 2048


def _conv_body(ps, pc, nf, wl, bl, wr, out):
    psv = ps[...]
    sums = psv[0] + psv[1]
    pcv = pc[...]
    cnt2 = pcv[0]
    for t in range(1, NW):
        cnt2 = cnt2 + pcv[t]
    inv2 = 1.0 / jnp.maximum(cnt2, 1.0)
    invb = jnp.broadcast_to(inv2[:, None, :], (_BN // 128, 128, 128))
    invb = invb.reshape(_BN, 128)
    rowm = lax.broadcasted_iota(jnp.int32, (_BN, 128), 0) & 127
    lane = lax.broadcasted_iota(jnp.int32, (_BN, 128), 1)
    invcol = jnp.sum(jnp.where(lane == rowm, invb, 0.0), axis=1,
                     keepdims=True)
    mean = sums * invcol
    r = _mm_t(mean, wl[...]) + _mm_t(nf[...], wr[...]) + bl[...]
    out[...] = jnp.maximum(r, 0.0)


def _tc_conv(psums, pcnts, nf, wl, bl, wr):
    bn = _BN
    return pl.pallas_call(
        _conv_body,
        grid=(pl.cdiv(NP2, bn),),
        in_specs=[
            pl.BlockSpec((NC, bn, D), lambda i: (0, i, 0)),
            pl.BlockSpec((NW, bn // 128, D), lambda i: (0, i, 0)),
            pl.BlockSpec((bn, D), lambda i: (i, 0)),
            pl.BlockSpec((H, D), lambda i: (0, 0)),
            pl.BlockSpec((1, H), lambda i: (0, 0)),
            pl.BlockSpec((H, D), lambda i: (0, 0)),
        ],
        out_specs=pl.BlockSpec((bn, H), lambda i: (i, 0)),
        out_shape=jax.ShapeDtypeStruct((NP2, H), jnp.float32),
    )(psums, pcnts, nf, wl, bl, wr)


def _mlp_body(x, y, wa, wb, b1, w2, b2, out):
    xv = x[...]
    yv = y[...]
    h = _mm_t(xv * yv, wa[...]) + _mm_t(xv - yv, wb[...]) + b1[...]
    h = jnp.maximum(h, 0.0)
    t = jnp.sum(h * w2[...], axis=1) + b2[0, 0]
    out[...] = 1.0 / (1.0 + jnp.exp(-t))


def _tc_mlp(x, y, wa, wb, b1, w2, b2):
    be = 2048
    return pl.pallas_call(
        _mlp_body,
        grid=(EP // be,),
        in_specs=[
            pl.BlockSpec((be, D), lambda i: (i, 0)),
            pl.BlockSpec((be, D), lambda i: (i, 0)),
            pl.BlockSpec((H, H), lambda i: (0, 0)),
            pl.BlockSpec((H, H), lambda i: (0, 0)),
            pl.BlockSpec((1, H), lambda i: (0, 0)),
            pl.BlockSpec((1, H), lambda i: (0, 0)),
            pl.BlockSpec(memory_space=pltpu.SMEM),
        ],
        out_specs=pl.BlockSpec((be,), lambda i: (i,)),
        out_shape=jax.ShapeDtypeStruct((EP,), jnp.float32),
    )(x, y, wa, wb, b1, w2, b2)


def kernel(node_features, edge_index, W_l, b_l, W_r, W_fc1, b_fc1, W_fc2, b_fc2):
    src = jnp.asarray(edge_index[0], jnp.int32)
    dst = jnp.asarray(edge_index[1], jnp.int32)
    pad = EP - E
    src = jnp.concatenate([src, jnp.zeros((pad,), jnp.int32)])
    dst = jnp.concatenate([dst, jnp.full((pad,), N, jnp.int32)])
    src = src.reshape(TOTCH, CHUNK)
    dst = dst.reshape(TOTCH, CHUNK)

    psums, pcnts = _sc_scatter(node_features, src, dst)
    out = _tc_conv(psums, pcnts, node_features,
                   W_l, b_l.reshape(1, H), W_r)
    x, y = _sc_gather(out, src, dst)

    wa = W_fc1[:, :H]
    wb = W_fc1[:, H:]
    p = _tc_mlp(x, y, wa, wb, b_fc1.reshape(1, H),
                W_fc2, b_fc2.reshape(1, 1))
    return p[:E].reshape(E, 1)


# speedup vs baseline: 1.0737x; 1.0737x over previous
import functools

import jax
import jax.numpy as jnp
from jax import lax
from jax.experimental import pallas as pl
from jax.experimental.pallas import tpu as pltpu
from jax.experimental.pallas import tpu_sc as plsc

N = 10000
E = 320000
D = 128
H = 128

NC = 2
NS = 16
NW = NC * NS
CHUNK = 128
NCHUNK = 80
EP = NW * NCHUNK * CHUNK
TOTCH = NW * NCHUNK
G0 = 120
G1 = 2 * NCHUNK - G0
GMAX = max(G0, G1)
NP2 = 10016
STRIPE = 624
TAIL0 = NS * STRIPE
ZR = 48
HR = 80
NPAD = HR * 128

_sc_mesh = plsc.VectorSubcoreMesh(core_axis_name="c", subcore_axis_name="s")
_sc_params = pltpu.CompilerParams(use_tc_tiling_on_sc=False,
                                  needs_layout_passes=False)


def _fill2d(ref, rows, cols, val):
    per_row = cols // 16

    def body(t, carry):
        i = t // per_row
        j = (t % per_row) * 16
        ref[i, pl.ds(j, 16)] = jnp.full((16,), val, ref.dtype)
        return carry

    lax.fori_loop(0, rows * per_row, body, 0)


def _fill1d(ref, n, val):
    def body(t, carry):
        ref[pl.ds(t * 16, 16)] = jnp.full((16,), val, ref.dtype)
        return carry

    lax.fori_loop(0, n // 16, body, 0)


@functools.partial(
    pl.kernel,
    out_type=(
        jax.ShapeDtypeStruct((NC, N, D), jnp.float32),
        jax.ShapeDtypeStruct((NW, HR, D), jnp.float32),
    ),
    mesh=_sc_mesh,
    compiler_params=_sc_params,
    scratch_types=(
        pltpu.VMEM((CHUNK,), jnp.int32),
        pltpu.VMEM((CHUNK,), jnp.int32),
        pltpu.VMEM((CHUNK,), jnp.int32),
        pltpu.VMEM((CHUNK,), jnp.int32),
        pltpu.VMEM((CHUNK, D), jnp.float32),
        pltpu.VMEM((CHUNK, D), jnp.float32),
        pltpu.VMEM((NPAD,), jnp.float32),
        pltpu.VMEM_SHARED((NP2, D), jnp.float32),
        pltpu.SemaphoreType.DMA,
        pltpu.SemaphoreType.DMA,
    ),
)
def _sc_scatter(nf, srci, dsti, sums_out, cnts_out,
                src_a, dst_a, src_b, dst_b, rows_a, rows_b,
                hist_v, sums_sh, sem_a, sem_b):
    c = lax.axis_index("c")
    s = lax.axis_index("s")
    wid = c * NS + s
    cbase = lax.select(c == 0, s * G0, NS * G0 + s * G1)
    nch = lax.select(c == 0, jnp.int32(G0), jnp.int32(G1))

    _fill2d(rows_a, CHUNK, D, 0.0)
    _fill1d(hist_v, NPAD, 0.0)

    row0 = s * STRIPE

    def zero_stripe(k, carry):
        pltpu.sync_copy(rows_a, sums_sh.at[pl.ds(row0 + k * CHUNK, CHUNK)])
        return carry

    lax.fori_loop(0, 4, zero_stripe, 0)
    pltpu.sync_copy(rows_a.at[pl.ds(0, STRIPE - 4 * CHUNK)],
                    sums_sh.at[pl.ds(row0 + 4 * CHUNK, STRIPE - 4 * CHUNK)])

    @pl.when(s == 0)
    def _zero_tail():
        pltpu.sync_copy(rows_a.at[pl.ds(0, 16)], sums_sh.at[pl.ds(TAIL0, 16)])

    plsc.subcore_barrier()

    ones16 = jnp.ones((16,), jnp.float32)

    pltpu.sync_copy(srci.at[cbase], src_a)
    pltpu.sync_copy(dsti.at[cbase], dst_a)
    pltpu.async_copy(nf.at[src_a], rows_a, sem_a)
    pltpu.sync_copy(srci.at[cbase + 1], src_b)
    pltpu.sync_copy(dsti.at[cbase + 1], dst_b)

    bufs = ((src_a, dst_a, rows_a, sem_a), (src_b, dst_b, rows_b, sem_b))

    def consume(j, cur, nxt):
        cur_s, cur_d, cur_rows, cur_sem = cur
        nxt_s, nxt_d, nxt_rows, nxt_sem = nxt

        @pl.when(j + 1 < nch)
        def _fire_next():
            pltpu.async_copy(nf.at[nxt_s], nxt_rows, nxt_sem)

        pltpu.make_async_copy(nf.at[cur_s], cur_rows, cur_sem).wait()
        pltpu.sync_copy(cur_rows, sums_sh.at[cur_d], add=True)
        for jj in range(CHUNK // 16):
            idx = cur_d[pl.ds(jj * 16, 16)]
            plsc.addupdate_scatter(hist_v, [idx], ones16)

        @pl.when(j + 2 < nch)
        def _prefetch_idx():
            pltpu.sync_copy(srci.at[cbase + j + 2], cur_s)
            pltpu.sync_copy(dsti.at[cbase + j + 2], cur_d)

    def body(k, carry):
        consume(2 * k, bufs[0], bufs[1])
        consume(2 * k + 1, bufs[1], bufs[0])
        return carry

    lax.fori_loop(0, nch // 2, body, 0)

    def repack(t, carry):
        rows_a[t // 8, pl.ds((t % 8) * 16, 16)] = hist_v[pl.ds(t * 16, 16)]
        return carry

    lax.fori_loop(0, NPAD // 16, repack, 0)
    plsc.subcore_barrier()

    pltpu.sync_copy(sums_sh.at[pl.ds(row0, STRIPE)],
                    sums_out.at[c, pl.ds(row0, STRIPE)])

    @pl.when(s == 0)
    def _dump_tail():
        pltpu.sync_copy(sums_sh.at[pl.ds(TAIL0, 16)],
                        sums_out.at[c, pl.ds(TAIL0, 16)])

    pltpu.sync_copy(rows_a.at[pl.ds(0, HR)], cnts_out.at[wid])


@functools.partial(
    pl.kernel,
    out_type=(
        jax.ShapeDtypeStruct((EP, D), jnp.float32),
        jax.ShapeDtypeStruct((EP, D), jnp.float32),
    ),
    mesh=_sc_mesh,
    compiler_params=_sc_params,
    scratch_types=(
        pltpu.VMEM((GMAX, CHUNK), jnp.int32),
        pltpu.VMEM((GMAX, CHUNK), jnp.int32),
        pltpu.VMEM((CHUNK, D), jnp.float32),
        pltpu.VMEM((CHUNK, D), jnp.float32),
        pltpu.VMEM((CHUNK, D), jnp.float32),
        pltpu.VMEM((CHUNK, D), jnp.float32),
        pltpu.SemaphoreType.DMA,
        pltpu.SemaphoreType.DMA,
        pltpu.SemaphoreType.DMA,
        pltpu.SemaphoreType.DMA,
    ),
)
def _sc_gather(table, srci, dsti, x_out, y_out,
               src_v, dst_v, x_a, y_a, x_b, y_b,
               gsem_a, gsem_b, wsem_a, wsem_b):
    c = lax.axis_index("c")
    s = lax.axis_index("s")
    cbase = lax.select(c == 0, s * G0, NS * G0 + s * G1)
    nch = lax.select(c == 0, jnp.int32(G0), jnp.int32(G1))
    base = cbase * CHUNK

    pltpu.sync_copy(srci.at[pl.ds(cbase, GMAX)], src_v)
    pltpu.sync_copy(dsti.at[pl.ds(cbase, GMAX)], dst_v)

    pltpu.async_copy(table.at[src_v.at[0]], x_a, gsem_a)
    pltpu.async_copy(table.at[dst_v.at[0]], y_a, gsem_a)

    bufs = ((x_a, y_a, gsem_a, wsem_a), (x_b, y_b, gsem_b, wsem_b))

    def consume(j, cur, nxt, first=False, last=False):
        cur_x, cur_y, cur_g, cur_w = cur
        nxt_x, nxt_y, nxt_g, nxt_w = nxt

        if not first:
            pltpu.make_async_copy(nxt_x, x_out.at[pl.ds(base, CHUNK)],
                                  nxt_w).wait()
            pltpu.make_async_copy(nxt_y, y_out.at[pl.ds(base, CHUNK)],
                                  nxt_w).wait()

        if not last:
            pltpu.async_copy(table.at[src_v.at[j + 1]], nxt_x, nxt_g)
            pltpu.async_copy(table.at[dst_v.at[j + 1]], nxt_y, nxt_g)

        pltpu.make_async_copy(table.at[src_v.at[0]], cur_x, cur_g).wait()
        pltpu.make_async_copy(table.at[dst_v.at[0]], cur_y, cur_g).wait()

        off = base + j * CHUNK
        pltpu.async_copy(cur_x, x_out.at[pl.ds(off, CHUNK)], cur_w)
        pltpu.async_copy(cur_y, y_out.at[pl.ds(off, CHUNK)], cur_w)

    consume(0, bufs[0], bufs[1], first=True)

    def body(k, carry):
        j = 2 * k + 1
        consume(j, bufs[1], bufs[0])
        consume(j + 1, bufs[0], bufs[1])
        return carry

    lax.fori_loop(0, (nch - 2) // 2, body, 0)

    consume(nch - 1, bufs[1], bufs[0], last=True)
    pltpu.make_async_copy(x_b, x_out.at[pl.ds(base, CHUNK)], wsem_b).wait()
    pltpu.make_async_copy(y_b, y_out.at[pl.ds(base, CHUNK)], wsem_b).wait()


def _mm_t(a, b):
    return lax.dot_general(a, b, (((1,), (1,)), ((), ())),
                           preferred_element_type=jnp.float32)


_BN = 2048


def _conv_body(ps, pc, nf, wl, bl, wr, out):
    psv = ps[...]
    sums = psv[0] + psv[1]
    pcv = pc[...]
    cnt2 = pcv[0]
    for t in range(1, NW):
        cnt2 = cnt2 + pcv[t]
    inv2 = 1.0 / jnp.maximum(cnt2, 1.0)
    invb = jnp.broadcast_to(inv2[:, None, :], (_BN // 128, 128, 128))
    invb = invb.reshape(_BN, 128)
    rowm = lax.broadcasted_iota(jnp.int32, (_BN, 128), 0) & 127
    lane = lax.broadcasted_iota(jnp.int32, (_BN, 128), 1)
    invcol = jnp.sum(jnp.where(lane == rowm, invb, 0.0), axis=1,
                     keepdims=True)
    mean = sums * invcol
    r = _mm_t(mean, wl[...]) + _mm_t(nf[...], wr[...]) + bl[...]
    out[...] = jnp.maximum(r, 0.0)


def _tc_conv(psums, pcnts, nf, wl, bl, wr):
    bn = _BN
    return pl.pallas_call(
        _conv_body,
        grid=(pl.cdiv(NP2, bn),),
        in_specs=[
            pl.BlockSpec((NC, bn, D), lambda i: (0, i, 0)),
            pl.BlockSpec((NW, bn // 128, D), lambda i: (0, i, 0)),
            pl.BlockSpec((bn, D), lambda i: (i, 0)),
            pl.BlockSpec((H, D), lambda i: (0, 0)),
            pl.BlockSpec((1, H), lambda i: (0, 0)),
            pl.BlockSpec((H, D), lambda i: (0, 0)),
        ],
        out_specs=pl.BlockSpec((bn, H), lambda i: (i, 0)),
        out_shape=jax.ShapeDtypeStruct((NP2, H), jnp.float32),
    )(psums, pcnts, nf, wl, bl, wr)


def _mlp_body(x, y, wa, wb, b1, w2, b2, out):
    xv = x[...]
    yv = y[...]
    h = _mm_t(xv * yv, wa[...]) + _mm_t(xv - yv, wb[...]) + b1[...]
    h = jnp.maximum(h, 0.0)
    t = jnp.sum(h * w2[...], axis=1) + b2[0, 0]
    out[...] = 1.0 / (1.0 + jnp.exp(-t))


def _tc_mlp(x, y, wa, wb, b1, w2, b2):
    be = 2048
    return pl.pallas_call(
        _mlp_body,
        grid=(EP // be,),
        in_specs=[
            pl.BlockSpec((be, D), lambda i: (i, 0)),
            pl.BlockSpec((be, D), lambda i: (i, 0)),
            pl.BlockSpec((H, H), lambda i: (0, 0)),
            pl.BlockSpec((H, H), lambda i: (0, 0)),
            pl.BlockSpec((1, H), lambda i: (0, 0)),
            pl.BlockSpec((1, H), lambda i: (0, 0)),
            pl.BlockSpec(memory_space=pltpu.SMEM),
        ],
        out_specs=pl.BlockSpec((be,), lambda i: (i,)),
        out_shape=jax.ShapeDtypeStruct((EP,), jnp.float32),
    )(x, y, wa, wb, b1, w2, b2)


def kernel(node_features, edge_index, W_l, b_l, W_r, W_fc1, b_fc1, W_fc2, b_fc2):
    src = jnp.asarray(edge_index[0], jnp.int32)
    dst = jnp.asarray(edge_index[1], jnp.int32)
    pad = EP - E
    src = jnp.concatenate([src, jnp.zeros((pad,), jnp.int32)])
    dst = jnp.concatenate([dst, jnp.full((pad,), N, jnp.int32)])
    src = src.reshape(TOTCH, CHUNK)
    dst = dst.reshape(TOTCH, CHUNK)

    psums, pcnts = _sc_scatter(node_features, src, dst)
    out = _tc_conv(psums, pcnts, node_features,
                   W_l, b_l.reshape(1, H), W_r)
    x, y = _sc_gather(out, src, dst)

    wa = W_fc1[:, :H]
    wb = W_fc1[:, H:]
    p = _tc_mlp(x, y, wa, wb, b_fc1.reshape(1, H),
                W_fc2, b_fc2.reshape(1, 1))
    return p[:E].reshape(E, 1)


# speedup vs baseline: 1.2192x; 1.1354x over previous
import functools

import jax
import jax.numpy as jnp
from jax import lax
from jax.experimental import pallas as pl
from jax.experimental.pallas import tpu as pltpu
from jax.experimental.pallas import tpu_sc as plsc

N = 10000
E = 320000
D = 128
H = 128

NC = 2
NS = 16
NW = NC * NS
CHUNK = 128
NCHUNK = 80
EP = NW * NCHUNK * CHUNK
TOTCH = NW * NCHUNK
G0 = 80
G1 = 2 * NCHUNK - G0
GMAX = max(G0, G1)
NP2 = 10016
STRIPE = 624
TAIL0 = NS * STRIPE
ZR = 48
HR = 80
NPAD = HR * 128

_sc_mesh = plsc.VectorSubcoreMesh(core_axis_name="c", subcore_axis_name="s")
_sc_params = pltpu.CompilerParams(use_tc_tiling_on_sc=False,
                                  needs_layout_passes=False)


def _fill2d(ref, rows, cols, val):
    per_row = cols // 16

    def body(t, carry):
        i = t // per_row
        j = (t % per_row) * 16
        ref[i, pl.ds(j, 16)] = jnp.full((16,), val, ref.dtype)
        return carry

    lax.fori_loop(0, rows * per_row, body, 0)


def _fill1d(ref, n, val):
    def body(t, carry):
        ref[pl.ds(t * 16, 16)] = jnp.full((16,), val, ref.dtype)
        return carry

    lax.fori_loop(0, n // 16, body, 0)


@functools.partial(
    pl.kernel,
    out_type=(
        jax.ShapeDtypeStruct((NC, N, D), jnp.float32),
        jax.ShapeDtypeStruct((NW, HR, D), jnp.float32),
    ),
    mesh=_sc_mesh,
    compiler_params=_sc_params,
    scratch_types=(
        pltpu.VMEM((CHUNK,), jnp.int32),
        pltpu.VMEM((CHUNK,), jnp.int32),
        pltpu.VMEM((CHUNK,), jnp.int32),
        pltpu.VMEM((CHUNK,), jnp.int32),
        pltpu.VMEM((CHUNK, D), jnp.float32),
        pltpu.VMEM((CHUNK, D), jnp.float32),
        pltpu.VMEM((NPAD,), jnp.float32),
        pltpu.VMEM_SHARED((NP2, D), jnp.float32),
        pltpu.SemaphoreType.DMA,
        pltpu.SemaphoreType.DMA,
    ),
)
def _sc_scatter(nf, srci, dsti, sums_out, cnts_out,
                src_a, dst_a, src_b, dst_b, rows_a, rows_b,
                hist_v, sums_sh, sem_a, sem_b):
    c = lax.axis_index("c")
    s = lax.axis_index("s")
    wid = c * NS + s
    cbase = lax.select(c == 0, s * G0, NS * G0 + s * G1)
    nch = lax.select(c == 0, jnp.int32(G0), jnp.int32(G1))

    _fill2d(rows_a, CHUNK, D, 0.0)
    _fill1d(hist_v, NPAD, 0.0)

    row0 = s * STRIPE

    def zero_stripe(k, carry):
        pltpu.sync_copy(rows_a, sums_sh.at[pl.ds(row0 + k * CHUNK, CHUNK)])
        return carry

    lax.fori_loop(0, 4, zero_stripe, 0)
    pltpu.sync_copy(rows_a.at[pl.ds(0, STRIPE - 4 * CHUNK)],
                    sums_sh.at[pl.ds(row0 + 4 * CHUNK, STRIPE - 4 * CHUNK)])

    @pl.when(s == 0)
    def _zero_tail():
        pltpu.sync_copy(rows_a.at[pl.ds(0, 16)], sums_sh.at[pl.ds(TAIL0, 16)])

    plsc.subcore_barrier()

    ones16 = jnp.ones((16,), jnp.float32)

    pltpu.sync_copy(srci.at[cbase], src_a)
    pltpu.sync_copy(dsti.at[cbase], dst_a)
    pltpu.async_copy(nf.at[src_a], rows_a, sem_a)
    pltpu.sync_copy(srci.at[cbase + 1], src_b)
    pltpu.sync_copy(dsti.at[cbase + 1], dst_b)

    bufs = ((src_a, dst_a, rows_a, sem_a), (src_b, dst_b, rows_b, sem_b))

    def consume(j, cur, nxt):
        cur_s, cur_d, cur_rows, cur_sem = cur
        nxt_s, nxt_d, nxt_rows, nxt_sem = nxt

        @pl.when(j + 1 < nch)
        def _fire_next():
            pltpu.async_copy(nf.at[nxt_s], nxt_rows, nxt_sem)

        pltpu.make_async_copy(nf.at[cur_s], cur_rows, cur_sem).wait()
        pltpu.sync_copy(cur_rows, sums_sh.at[cur_d], add=True)
        for jj in range(CHUNK // 16):
            idx = cur_d[pl.ds(jj * 16, 16)]
            plsc.addupdate_scatter(hist_v, [idx], ones16)

        @pl.when(j + 2 < nch)
        def _prefetch_idx():
            pltpu.sync_copy(srci.at[cbase + j + 2], cur_s)
            pltpu.sync_copy(dsti.at[cbase + j + 2], cur_d)

    def body(k, carry):
        consume(2 * k, bufs[0], bufs[1])
        consume(2 * k + 1, bufs[1], bufs[0])
        return carry

    lax.fori_loop(0, nch // 2, body, 0)

    def repack(t, carry):
        rows_a[t // 8, pl.ds((t % 8) * 16, 16)] = hist_v[pl.ds(t * 16, 16)]
        return carry

    lax.fori_loop(0, NPAD // 16, repack, 0)
    plsc.subcore_barrier()

    pltpu.sync_copy(sums_sh.at[pl.ds(row0, STRIPE)],
                    sums_out.at[c, pl.ds(row0, STRIPE)])

    @pl.when(s == 0)
    def _dump_tail():
        pltpu.sync_copy(sums_sh.at[pl.ds(TAIL0, 16)],
                        sums_out.at[c, pl.ds(TAIL0, 16)])

    pltpu.sync_copy(rows_a.at[pl.ds(0, HR)], cnts_out.at[wid])


@functools.partial(
    pl.kernel,
    out_type=(
        jax.ShapeDtypeStruct((EP, D), jnp.float32),
        jax.ShapeDtypeStruct((EP, D), jnp.float32),
    ),
    mesh=_sc_mesh,
    compiler_params=_sc_params,
    scratch_types=(
        pltpu.VMEM((GMAX, CHUNK), jnp.int32),
        pltpu.VMEM((GMAX, CHUNK), jnp.int32),
        pltpu.VMEM((CHUNK, D), jnp.float32),
        pltpu.VMEM((CHUNK, D), jnp.float32),
        pltpu.VMEM((CHUNK, D), jnp.float32),
        pltpu.VMEM((CHUNK, D), jnp.float32),
        pltpu.SemaphoreType.DMA,
        pltpu.SemaphoreType.DMA,
        pltpu.SemaphoreType.DMA,
        pltpu.SemaphoreType.DMA,
    ),
)
def _sc_gather(table, srci, dsti, x_out, y_out,
               src_v, dst_v, x_a, y_a, x_b, y_b,
               gsem_a, gsem_b, wsem_a, wsem_b):
    c = lax.axis_index("c")
    s = lax.axis_index("s")
    cbase = lax.select(c == 0, s * G0, NS * G0 + s * G1)
    nch = lax.select(c == 0, jnp.int32(G0), jnp.int32(G1))
    base = cbase * CHUNK

    pltpu.sync_copy(srci.at[pl.ds(cbase, GMAX)], src_v)
    pltpu.sync_copy(dsti.at[pl.ds(cbase, GMAX)], dst_v)

    pltpu.async_copy(table.at[src_v.at[0]], x_a, gsem_a)
    pltpu.async_copy(table.at[dst_v.at[0]], y_a, gsem_a)

    bufs = ((x_a, y_a, gsem_a, wsem_a), (x_b, y_b, gsem_b, wsem_b))

    def consume(j, cur, nxt, first=False, last=False):
        cur_x, cur_y, cur_g, cur_w = cur
        nxt_x, nxt_y, nxt_g, nxt_w = nxt

        if not first:
            pltpu.make_async_copy(nxt_x, x_out.at[pl.ds(base, CHUNK)],
                                  nxt_w).wait()
            pltpu.make_async_copy(nxt_y, y_out.at[pl.ds(base, CHUNK)],
                                  nxt_w).wait()

        if not last:
            pltpu.async_copy(table.at[src_v.at[j + 1]], nxt_x, nxt_g)
            pltpu.async_copy(table.at[dst_v.at[j + 1]], nxt_y, nxt_g)

        pltpu.make_async_copy(table.at[src_v.at[0]], cur_x, cur_g).wait()
        pltpu.make_async_copy(table.at[dst_v.at[0]], cur_y, cur_g).wait()

        off = base + j * CHUNK
        pltpu.async_copy(cur_x, x_out.at[pl.ds(off, CHUNK)], cur_w)
        pltpu.async_copy(cur_y, y_out.at[pl.ds(off, CHUNK)], cur_w)

    consume(0, bufs[0], bufs[1], first=True)

    def body(k, carry):
        j = 2 * k + 1
        consume(j, bufs[1], bufs[0])
        consume(j + 1, bufs[0], bufs[1])
        return carry

    lax.fori_loop(0, (nch - 2) // 2, body, 0)

    consume(nch - 1, bufs[1], bufs[0], last=True)
    pltpu.make_async_copy(x_b, x_out.at[pl.ds(base, CHUNK)], wsem_b).wait()
    pltpu.make_async_copy(y_b, y_out.at[pl.ds(base, CHUNK)], wsem_b).wait()


def _mm_t(a, b):
    return lax.dot_general(a, b, (((1,), (1,)), ((), ())),
                           preferred_element_type=jnp.float32)


_BN = 2048


def _conv_body(ps, pc, nf, wl, bl, wr, out):
    psv = ps[...]
    sums = psv[0] + psv[1]
    pcv = pc[...]
    cnt2 = pcv[0]
    for t in range(1, NW):
        cnt2 = cnt2 + pcv[t]
    inv2 = 1.0 / jnp.maximum(cnt2, 1.0)
    invb = jnp.broadcast_to(inv2[:, None, :], (_BN // 128, 128, 128))
    invb = invb.reshape(_BN, 128)
    rowm = lax.broadcasted_iota(jnp.int32, (_BN, 128), 0) & 127
    lane = lax.broadcasted_iota(jnp.int32, (_BN, 128), 1)
    invcol = jnp.sum(jnp.where(lane == rowm, invb, 0.0), axis=1,
                     keepdims=True)
    mean = sums * invcol
    r = _mm_t(mean, wl[...]) + _mm_t(nf[...], wr[...]) + bl[...]
    out[...] = jnp.maximum(r, 0.0)


def _tc_conv(psums, pcnts, nf, wl, bl, wr):
    bn = _BN
    return pl.pallas_call(
        _conv_body,
        grid=(pl.cdiv(NP2, bn),),
        in_specs=[
            pl.BlockSpec((NC, bn, D), lambda i: (0, i, 0)),
            pl.BlockSpec((NW, bn // 128, D), lambda i: (0, i, 0)),
            pl.BlockSpec((bn, D), lambda i: (i, 0)),
            pl.BlockSpec((H, D), lambda i: (0, 0)),
            pl.BlockSpec((1, H), lambda i: (0, 0)),
            pl.BlockSpec((H, D), lambda i: (0, 0)),
        ],
        out_specs=pl.BlockSpec((bn, H), lambda i: (i, 0)),
        out_shape=jax.ShapeDtypeStruct((NP2, H), jnp.float32),
    )(psums, pcnts, nf, wl, bl, wr)


def _mlp_body(x, y, wa, wb, b1, w2, b2, out):
    xv = x[...]
    yv = y[...]
    h = _mm_t(xv * yv, wa[...]) + _mm_t(xv - yv, wb[...]) + b1[...]
    h = jnp.maximum(h, 0.0)
    t = _mm_t(w2[...], h) + b2[0, 0]
    out[...] = (1.0 / (1.0 + jnp.exp(-t))).reshape(t.shape[1])


def _tc_mlp(x, y, wa, wb, b1, w2, b2):
    be = 2048
    return pl.pallas_call(
        _mlp_body,
        grid=(EP // be,),
        in_specs=[
            pl.BlockSpec((be, D), lambda i: (i, 0)),
            pl.BlockSpec((be, D), lambda i: (i, 0)),
            pl.BlockSpec((H, H), lambda i: (0, 0)),
            pl.BlockSpec((H, H), lambda i: (0, 0)),
            pl.BlockSpec((1, H), lambda i: (0, 0)),
            pl.BlockSpec((1, H), lambda i: (0, 0)),
            pl.BlockSpec(memory_space=pltpu.SMEM),
        ],
        out_specs=pl.BlockSpec((be,), lambda i: (i,)),
        out_shape=jax.ShapeDtypeStruct((EP,), jnp.float32),
    )(x, y, wa, wb, b1, w2, b2)


def kernel(node_features, edge_index, W_l, b_l, W_r, W_fc1, b_fc1, W_fc2, b_fc2):
    src = jnp.asarray(edge_index[0], jnp.int32)
    dst = jnp.asarray(edge_index[1], jnp.int32)
    pad = EP - E
    src = jnp.concatenate([src, jnp.zeros((pad,), jnp.int32)])
    dst = jnp.concatenate([dst, jnp.full((pad,), N, jnp.int32)])
    src = src.reshape(TOTCH, CHUNK)
    dst = dst.reshape(TOTCH, CHUNK)

    psums, pcnts = _sc_scatter(node_features, src, dst)
    out = _tc_conv(psums, pcnts, node_features,
                   W_l, b_l.reshape(1, H), W_r)
    x, y = _sc_gather(out, src, dst)

    wa = W_fc1[:, :H]
    wb = W_fc1[:, H:]
    p = _tc_mlp(x, y, wa, wb, b_fc1.reshape(1, H),
                W_fc2, b_fc2.reshape(1, 1))
    return p[:E].reshape(E, 1)


# speedup vs baseline: 1.9491x; 1.5987x over previous
import functools

import jax
import jax.numpy as jnp
from jax import lax
from jax.experimental import pallas as pl
from jax.experimental.pallas import tpu as pltpu
from jax.experimental.pallas import tpu_sc as plsc

N = 10000
E = 320000
D = 128
H = 128

NC = 2
NS = 16
NW = NC * NS
CHUNK = 128
NCHUNK = 80
EP = NW * NCHUNK * CHUNK
TOTCH = NW * NCHUNK
G0 = 80
G1 = 2 * NCHUNK - G0
GMAX = max(G0, G1)
NP2 = 10016
STRIPE = 624
TAIL0 = NS * STRIPE
ZR = 48
HR = 80
NPAD = HR * 128

_sc_mesh = plsc.VectorSubcoreMesh(core_axis_name="c", subcore_axis_name="s")
_sc_params = pltpu.CompilerParams(use_tc_tiling_on_sc=False,
                                  needs_layout_passes=False)


def _fill2d(ref, rows, cols, val):
    per_row = cols // 16

    def body(t, carry):
        i = t // per_row
        j = (t % per_row) * 16
        ref[i, pl.ds(j, 16)] = jnp.full((16,), val, ref.dtype)
        return carry

    lax.fori_loop(0, rows * per_row, body, 0)


def _fill1d(ref, n, val):
    def body(t, carry):
        ref[pl.ds(t * 16, 16)] = jnp.full((16,), val, ref.dtype)
        return carry

    lax.fori_loop(0, n // 16, body, 0)


@functools.partial(
    pl.kernel,
    out_type=(
        jax.ShapeDtypeStruct((NC, N, D), jnp.float32),
        jax.ShapeDtypeStruct((NW, HR, D), jnp.float32),
    ),
    mesh=_sc_mesh,
    compiler_params=_sc_params,
    scratch_types=(
        pltpu.VMEM((CHUNK,), jnp.int32),
        pltpu.VMEM((CHUNK,), jnp.int32),
        pltpu.VMEM((CHUNK,), jnp.int32),
        pltpu.VMEM((CHUNK,), jnp.int32),
        pltpu.VMEM((CHUNK, D), jnp.float32),
        pltpu.VMEM((CHUNK, D), jnp.float32),
        pltpu.VMEM((NPAD,), jnp.float32),
        pltpu.VMEM_SHARED((NP2, D), jnp.float32),
        pltpu.SemaphoreType.DMA,
        pltpu.SemaphoreType.DMA,
    ),
)
def _sc_scatter(nf, srci, dsti, sums_out, cnts_out,
                src_a, dst_a, src_b, dst_b, rows_a, rows_b,
                hist_v, sums_sh, sem_a, sem_b):
    c = lax.axis_index("c")
    s = lax.axis_index("s")
    wid = c * NS + s
    cbase = lax.select(c == 0, s * G0, NS * G0 + s * G1)
    nch = lax.select(c == 0, jnp.int32(G0), jnp.int32(G1))

    _fill2d(rows_a, CHUNK, D, 0.0)
    _fill1d(hist_v, NPAD, 0.0)

    row0 = s * STRIPE

    def zero_stripe(k, carry):
        pltpu.sync_copy(rows_a, sums_sh.at[pl.ds(row0 + k * CHUNK, CHUNK)])
        return carry

    lax.fori_loop(0, 4, zero_stripe, 0)
    pltpu.sync_copy(rows_a.at[pl.ds(0, STRIPE - 4 * CHUNK)],
                    sums_sh.at[pl.ds(row0 + 4 * CHUNK, STRIPE - 4 * CHUNK)])

    @pl.when(s == 0)
    def _zero_tail():
        pltpu.sync_copy(rows_a.at[pl.ds(0, 16)], sums_sh.at[pl.ds(TAIL0, 16)])

    plsc.subcore_barrier()

    ones16 = jnp.ones((16,), jnp.float32)

    pltpu.sync_copy(srci.at[cbase], src_a)
    pltpu.sync_copy(dsti.at[cbase], dst_a)
    pltpu.async_copy(nf.at[src_a], rows_a, sem_a)
    pltpu.sync_copy(srci.at[cbase + 1], src_b)
    pltpu.sync_copy(dsti.at[cbase + 1], dst_b)

    bufs = ((src_a, dst_a, rows_a, sem_a), (src_b, dst_b, rows_b, sem_b))

    def consume(j, cur, nxt):
        cur_s, cur_d, cur_rows, cur_sem = cur
        nxt_s, nxt_d, nxt_rows, nxt_sem = nxt

        @pl.when(j + 1 < nch)
        def _fire_next():
            pltpu.async_copy(nf.at[nxt_s], nxt_rows, nxt_sem)

        pltpu.make_async_copy(nf.at[cur_s], cur_rows, cur_sem).wait()
        pltpu.sync_copy(cur_rows, sums_sh.at[cur_d], add=True)
        for jj in range(CHUNK // 16):
            idx = cur_d[pl.ds(jj * 16, 16)]
            plsc.addupdate_scatter(hist_v, [idx], ones16)

        @pl.when(j + 2 < nch)
        def _prefetch_idx():
            pltpu.sync_copy(srci.at[cbase + j + 2], cur_s)
            pltpu.sync_copy(dsti.at[cbase + j + 2], cur_d)

    def body(k, carry):
        consume(2 * k, bufs[0], bufs[1])
        consume(2 * k + 1, bufs[1], bufs[0])
        return carry

    lax.fori_loop(0, nch // 2, body, 0)

    def repack(t, carry):
        rows_a[t // 8, pl.ds((t % 8) * 16, 16)] = hist_v[pl.ds(t * 16, 16)]
        return carry

    lax.fori_loop(0, NPAD // 16, repack, 0)
    plsc.subcore_barrier()

    pltpu.sync_copy(sums_sh.at[pl.ds(row0, STRIPE)],
                    sums_out.at[c, pl.ds(row0, STRIPE)])

    @pl.when(s == 0)
    def _dump_tail():
        pltpu.sync_copy(sums_sh.at[pl.ds(TAIL0, 16)],
                        sums_out.at[c, pl.ds(TAIL0, 16)])

    pltpu.sync_copy(rows_a.at[pl.ds(0, HR)], cnts_out.at[wid])


@functools.partial(
    pl.kernel,
    out_type=(
        jax.ShapeDtypeStruct((EP, D), jnp.float32),
        jax.ShapeDtypeStruct((EP, D), jnp.float32),
    ),
    mesh=_sc_mesh,
    compiler_params=_sc_params,
    scratch_types=(
        pltpu.VMEM((CHUNK,), jnp.int32),
        pltpu.VMEM((CHUNK,), jnp.int32),
        pltpu.VMEM((CHUNK, D), jnp.float32),
        pltpu.VMEM((CHUNK, D), jnp.float32),
        pltpu.VMEM_SHARED((NP2, D), jnp.float32),
        pltpu.SemaphoreType.DMA,
        pltpu.SemaphoreType.DMA,
        pltpu.SemaphoreType.DMA,
        pltpu.SemaphoreType.DMA,
        pltpu.SemaphoreType.DMA,
        pltpu.SemaphoreType.DMA,
    ),
)
def _sc_gather(table, srci, dsti, x_out, y_out,
               ia, ib, rows_a, rows_b, table_sh,
               gsem_a, gsem_b, wsem_a, wsem_b, isem_a, isem_b):
    c = lax.axis_index("c")
    s = lax.axis_index("s")
    wid = c * NS + s
    cbase = wid * NCHUNK
    base = cbase * CHUNK

    row0 = s * STRIPE
    pltpu.sync_copy(table.at[pl.ds(row0, STRIPE)],
                    table_sh.at[pl.ds(row0, STRIPE)])

    @pl.when(s == 0)
    def _stage_tail():
        pltpu.sync_copy(table.at[pl.ds(TAIL0, NP2 - TAIL0)],
                        table_sh.at[pl.ds(TAIL0, NP2 - TAIL0)])

    plsc.subcore_barrier()

    pltpu.sync_copy(srci.at[cbase], ia)
    pltpu.async_copy(table_sh.at[ia], rows_a, gsem_a)
    pltpu.async_copy(dsti.at[cbase], ib, isem_b)

    def slot_x(k):
        @pl.when(k >= 1)
        def _drain_prev_write():
            pltpu.make_async_copy(rows_b, y_out.at[pl.ds(base, CHUNK)],
                                  wsem_b).wait()

        pltpu.make_async_copy(dsti.at[cbase], ib, isem_b).wait()
        pltpu.async_copy(table_sh.at[ib], rows_b, gsem_b)

        pltpu.make_async_copy(table.at[pl.ds(0, CHUNK)], rows_a,
                              gsem_a).wait()

        @pl.when(k + 1 < NCHUNK)
        def _fire_idx():
            pltpu.async_copy(srci.at[cbase + k + 1], ia, isem_a)

        pltpu.async_copy(rows_a, x_out.at[pl.ds(base + k * CHUNK, CHUNK)],
                         wsem_a)

    def slot_y(k):
        pltpu.make_async_copy(rows_a, x_out.at[pl.ds(base, CHUNK)],
                              wsem_a).wait()

        @pl.when(k + 1 < NCHUNK)
        def _fire_next():
            pltpu.make_async_copy(srci.at[cbase], ia, isem_a).wait()
            pltpu.async_copy(table_sh.at[ia], rows_a, gsem_a)

        pltpu.make_async_copy(table.at[pl.ds(0, CHUNK)], rows_b,
                              gsem_b).wait()

        @pl.when(k + 1 < NCHUNK)
        def _fire_idx():
            pltpu.async_copy(dsti.at[cbase + k + 1], ib, isem_b)

        pltpu.async_copy(rows_b, y_out.at[pl.ds(base + k * CHUNK, CHUNK)],
                         wsem_b)

    def body(k, carry):
        slot_x(k)
        slot_y(k)
        return carry

    lax.fori_loop(0, NCHUNK, body, 0)

    pltpu.make_async_copy(rows_b, y_out.at[pl.ds(base, CHUNK)], wsem_b).wait()


def _mm_t(a, b):
    return lax.dot_general(a, b, (((1,), (1,)), ((), ())),
                           preferred_element_type=jnp.float32)


_BN = 2048


def _conv_body(ps, pc, nf, wl, bl, wr, out):
    psv = ps[...]
    sums = psv[0] + psv[1]
    pcv = pc[...]
    cnt2 = pcv[0]
    for t in range(1, NW):
        cnt2 = cnt2 + pcv[t]
    inv2 = 1.0 / jnp.maximum(cnt2, 1.0)
    invb = jnp.broadcast_to(inv2[:, None, :], (_BN // 128, 128, 128))
    invb = invb.reshape(_BN, 128)
    rowm = lax.broadcasted_iota(jnp.int32, (_BN, 128), 0) & 127
    lane = lax.broadcasted_iota(jnp.int32, (_BN, 128), 1)
    invcol = jnp.sum(jnp.where(lane == rowm, invb, 0.0), axis=1,
                     keepdims=True)
    mean = sums * invcol
    r = _mm_t(mean, wl[...]) + _mm_t(nf[...], wr[...]) + bl[...]
    out[...] = jnp.maximum(r, 0.0)


def _tc_conv(psums, pcnts, nf, wl, bl, wr):
    bn = _BN
    return pl.pallas_call(
        _conv_body,
        grid=(pl.cdiv(NP2, bn),),
        in_specs=[
            pl.BlockSpec((NC, bn, D), lambda i: (0, i, 0)),
            pl.BlockSpec((NW, bn // 128, D), lambda i: (0, i, 0)),
            pl.BlockSpec((bn, D), lambda i: (i, 0)),
            pl.BlockSpec((H, D), lambda i: (0, 0)),
            pl.BlockSpec((1, H), lambda i: (0, 0)),
            pl.BlockSpec((H, D), lambda i: (0, 0)),
        ],
        out_specs=pl.BlockSpec((bn, H), lambda i: (i, 0)),
        out_shape=jax.ShapeDtypeStruct((NP2, H), jnp.float32),
    )(psums, pcnts, nf, wl, bl, wr)


def _mlp_body(x, y, wa, wb, b1, w2, b2, out):
    xv = x[...]
    yv = y[...]
    h = _mm_t(xv * yv, wa[...]) + _mm_t(xv - yv, wb[...]) + b1[...]
    h = jnp.maximum(h, 0.0)
    t = _mm_t(w2[...], h) + b2[0, 0]
    out[...] = (1.0 / (1.0 + jnp.exp(-t))).reshape(t.shape[1])


def _tc_mlp(x, y, wa, wb, b1, w2, b2):
    be = 2048
    return pl.pallas_call(
        _mlp_body,
        grid=(EP // be,),
        in_specs=[
            pl.BlockSpec((be, D), lambda i: (i, 0)),
            pl.BlockSpec((be, D), lambda i: (i, 0)),
            pl.BlockSpec((H, H), lambda i: (0, 0)),
            pl.BlockSpec((H, H), lambda i: (0, 0)),
            pl.BlockSpec((1, H), lambda i: (0, 0)),
            pl.BlockSpec((1, H), lambda i: (0, 0)),
            pl.BlockSpec(memory_space=pltpu.SMEM),
        ],
        out_specs=pl.BlockSpec((be,), lambda i: (i,)),
        out_shape=jax.ShapeDtypeStruct((EP,), jnp.float32),
    )(x, y, wa, wb, b1, w2, b2)


def kernel(node_features, edge_index, W_l, b_l, W_r, W_fc1, b_fc1, W_fc2, b_fc2):
    src = jnp.asarray(edge_index[0], jnp.int32)
    dst = jnp.asarray(edge_index[1], jnp.int32)
    pad = EP - E
    src = jnp.concatenate([src, jnp.zeros((pad,), jnp.int32)])
    dst = jnp.concatenate([dst, jnp.full((pad,), N, jnp.int32)])
    src = src.reshape(TOTCH, CHUNK)
    dst = dst.reshape(TOTCH, CHUNK)

    psums, pcnts = _sc_scatter(node_features, src, dst)
    out = _tc_conv(psums, pcnts, node_features,
                   W_l, b_l.reshape(1, H), W_r)
    x, y = _sc_gather(out, src, dst)

    wa = W_fc1[:, :H]
    wb = W_fc1[:, H:]
    p = _tc_mlp(x, y, wa, wb, b_fc1.reshape(1, H),
                W_fc2, b_fc2.reshape(1, 1))
    return p[:E].reshape(E, 1)


# speedup vs baseline: 2.5220x; 1.2939x over previous
import functools

import jax
import jax.numpy as jnp
from jax import lax
from jax.experimental import pallas as pl
from jax.experimental.pallas import tpu as pltpu
from jax.experimental.pallas import tpu_sc as plsc

N = 10000
E = 320000
D = 128
H = 128

NC = 2
NS = 16
NW = NC * NS
CHUNK = 128
NCHUNK = 80
EP = NW * NCHUNK * CHUNK
TOTCH = NW * NCHUNK
G0 = 80
G1 = 2 * NCHUNK - G0
GMAX = max(G0, G1)
NP2 = 10016
STRIPE = 624
TAIL0 = NS * STRIPE
ZR = 48
HR = 80
NPAD = HR * 128

_sc_mesh = plsc.VectorSubcoreMesh(core_axis_name="c", subcore_axis_name="s")
_sc_params = pltpu.CompilerParams(use_tc_tiling_on_sc=False,
                                  needs_layout_passes=False)


def _fill2d(ref, rows, cols, val):
    per_row = cols // 16

    def body(t, carry):
        i = t // per_row
        j = (t % per_row) * 16
        ref[i, pl.ds(j, 16)] = jnp.full((16,), val, ref.dtype)
        return carry

    lax.fori_loop(0, rows * per_row, body, 0)


def _fill1d(ref, n, val):
    def body(t, carry):
        ref[pl.ds(t * 16, 16)] = jnp.full((16,), val, ref.dtype)
        return carry

    lax.fori_loop(0, n // 16, body, 0)


DH = D // NC
TPC = TOTCH // NS


@functools.partial(
    pl.kernel,
    out_type=(
        jax.ShapeDtypeStruct((N, D), jnp.float32),
        jax.ShapeDtypeStruct((NS, HR, D), jnp.float32),
    ),
    mesh=_sc_mesh,
    compiler_params=_sc_params,
    scratch_types=(
        pltpu.VMEM((CHUNK,), jnp.int32),
        pltpu.VMEM((CHUNK,), jnp.int32),
        pltpu.VMEM((CHUNK,), jnp.int32),
        pltpu.VMEM((CHUNK,), jnp.int32),
        pltpu.VMEM((CHUNK, DH), jnp.float32),
        pltpu.VMEM((CHUNK, DH), jnp.float32),
        pltpu.VMEM((NPAD,), jnp.float32),
        pltpu.VMEM((HR, D), jnp.float32),
        pltpu.VMEM_SHARED((N, DH), jnp.float32),
        pltpu.VMEM_SHARED((NP2, DH), jnp.float32),
        pltpu.SemaphoreType.DMA,
        pltpu.SemaphoreType.DMA,
    ),
)
def _sc_scatter(nf, srci, dsti, sums_out, cnts_out,
                src_a, dst_a, src_b, dst_b, rows_a, rows_b,
                hist_v, hist2_v, nf_sh, sums_sh, sem_a, sem_b):
    c = lax.axis_index("c")
    s = lax.axis_index("s")
    col0 = c * DH

    _fill2d(rows_a, CHUNK, DH, 0.0)
    _fill1d(hist_v, NPAD, 0.0)

    row0 = s * STRIPE

    pltpu.sync_copy(nf.at[pl.ds(row0, STRIPE), pl.ds(col0, DH)],
                    nf_sh.at[pl.ds(row0, STRIPE)])

    def zero_stripe(k, carry):
        pltpu.sync_copy(rows_a, sums_sh.at[pl.ds(row0 + k * CHUNK, CHUNK)])
        return carry

    lax.fori_loop(0, 4, zero_stripe, 0)
    pltpu.sync_copy(rows_a.at[pl.ds(0, STRIPE - 4 * CHUNK)],
                    sums_sh.at[pl.ds(row0 + 4 * CHUNK, STRIPE - 4 * CHUNK)])

    @pl.when(s == 0)
    def _stage_tail():
        pltpu.sync_copy(nf.at[pl.ds(TAIL0, N - TAIL0), pl.ds(col0, DH)],
                        nf_sh.at[pl.ds(TAIL0, N - TAIL0)])
        pltpu.sync_copy(rows_a.at[pl.ds(0, 16)], sums_sh.at[pl.ds(TAIL0, 16)])

    plsc.subcore_barrier()

    ones16 = jnp.ones((16,), jnp.float32)
    cbase = s * TPC

    pltpu.sync_copy(srci.at[cbase], src_a)
    pltpu.sync_copy(dsti.at[cbase], dst_a)
    pltpu.async_copy(nf_sh.at[src_a], rows_a, sem_a)
    pltpu.sync_copy(srci.at[cbase + 1], src_b)
    pltpu.sync_copy(dsti.at[cbase + 1], dst_b)

    bufs = ((src_a, dst_a, rows_a, sem_a), (src_b, dst_b, rows_b, sem_b))

    def consume(j, cur, nxt):
        cur_s, cur_d, cur_rows, cur_sem = cur
        nxt_s, nxt_d, nxt_rows, nxt_sem = nxt

        @pl.when(j + 1 < TPC)
        def _fire_next():
            pltpu.async_copy(nf_sh.at[nxt_s], nxt_rows, nxt_sem)

        pltpu.make_async_copy(nf.at[pl.ds(0, CHUNK), pl.ds(col0, DH)],
                              cur_rows, cur_sem).wait()
        pltpu.sync_copy(cur_rows, sums_sh.at[cur_d], add=True)

        @pl.when(c == 0)
        def _hist():
            for jj in range(CHUNK // 16):
                idx = cur_d[pl.ds(jj * 16, 16)]
                plsc.addupdate_scatter(hist_v, [idx], ones16)

        @pl.when(j + 2 < TPC)
        def _prefetch_idx():
            pltpu.sync_copy(srci.at[cbase + j + 2], cur_s)
            pltpu.sync_copy(dsti.at[cbase + j + 2], cur_d)

    def body(k, carry):
        consume(2 * k, bufs[0], bufs[1])
        consume(2 * k + 1, bufs[1], bufs[0])
        return carry

    lax.fori_loop(0, TPC // 2, body, 0)

    def repack(t, carry):
        hist2_v[t // 8, pl.ds((t % 8) * 16, 16)] = hist_v[pl.ds(t * 16, 16)]
        return carry

    lax.fori_loop(0, NPAD // 16, repack, 0)
    plsc.subcore_barrier()

    pltpu.sync_copy(sums_sh.at[pl.ds(row0, STRIPE)],
                    sums_out.at[pl.ds(row0, STRIPE), pl.ds(col0, DH)])

    @pl.when(s == 0)
    def _dump_tail():
        pltpu.sync_copy(sums_sh.at[pl.ds(TAIL0, N - TAIL0)],
                        sums_out.at[pl.ds(TAIL0, N - TAIL0), pl.ds(col0, DH)])

    @pl.when(c == 0)
    def _dump_hist():
        pltpu.sync_copy(hist2_v, cnts_out.at[s])


@functools.partial(
    pl.kernel,
    out_type=(
        jax.ShapeDtypeStruct((EP, D), jnp.float32),
        jax.ShapeDtypeStruct((EP, D), jnp.float32),
    ),
    mesh=_sc_mesh,
    compiler_params=_sc_params,
    scratch_types=(
        pltpu.VMEM((CHUNK,), jnp.int32),
        pltpu.VMEM((CHUNK,), jnp.int32),
        pltpu.VMEM((CHUNK, D), jnp.float32),
        pltpu.VMEM((CHUNK, D), jnp.float32),
        pltpu.VMEM_SHARED((NP2, D), jnp.float32),
        pltpu.SemaphoreType.DMA,
        pltpu.SemaphoreType.DMA,
        pltpu.SemaphoreType.DMA,
        pltpu.SemaphoreType.DMA,
        pltpu.SemaphoreType.DMA,
        pltpu.SemaphoreType.DMA,
    ),
)
def _sc_gather(table, srci, dsti, x_out, y_out,
               ia, ib, rows_a, rows_b, table_sh,
               gsem_a, gsem_b, wsem_a, wsem_b, isem_a, isem_b):
    c = lax.axis_index("c")
    s = lax.axis_index("s")
    wid = c * NS + s
    cbase = wid * NCHUNK
    base = cbase * CHUNK

    row0 = s * STRIPE
    pltpu.sync_copy(table.at[pl.ds(row0, STRIPE)],
                    table_sh.at[pl.ds(row0, STRIPE)])

    @pl.when(s == 0)
    def _stage_tail():
        pltpu.sync_copy(table.at[pl.ds(TAIL0, NP2 - TAIL0)],
                        table_sh.at[pl.ds(TAIL0, NP2 - TAIL0)])

    plsc.subcore_barrier()

    pltpu.sync_copy(srci.at[cbase], ia)
    pltpu.async_copy(table_sh.at[ia], rows_a, gsem_a)
    pltpu.async_copy(dsti.at[cbase], ib, isem_b)

    def slot_x(k):
        @pl.when(k >= 1)
        def _drain_prev_write():
            pltpu.make_async_copy(rows_b, y_out.at[pl.ds(base, CHUNK)],
                                  wsem_b).wait()

        pltpu.make_async_copy(dsti.at[cbase], ib, isem_b).wait()
        pltpu.async_copy(table_sh.at[ib], rows_b, gsem_b)

        pltpu.make_async_copy(table.at[pl.ds(0, CHUNK)], rows_a,
                              gsem_a).wait()

        @pl.when(k + 1 < NCHUNK)
        def _fire_idx():
            pltpu.async_copy(srci.at[cbase + k + 1], ia, isem_a)

        pltpu.async_copy(rows_a, x_out.at[pl.ds(base + k * CHUNK, CHUNK)],
                         wsem_a)

    def slot_y(k):
        pltpu.make_async_copy(rows_a, x_out.at[pl.ds(base, CHUNK)],
                              wsem_a).wait()

        @pl.when(k + 1 < NCHUNK)
        def _fire_next():
            pltpu.make_async_copy(srci.at[cbase], ia, isem_a).wait()
            pltpu.async_copy(table_sh.at[ia], rows_a, gsem_a)

        pltpu.make_async_copy(table.at[pl.ds(0, CHUNK)], rows_b,
                              gsem_b).wait()

        @pl.when(k + 1 < NCHUNK)
        def _fire_idx():
            pltpu.async_copy(dsti.at[cbase + k + 1], ib, isem_b)

        pltpu.async_copy(rows_b, y_out.at[pl.ds(base + k * CHUNK, CHUNK)],
                         wsem_b)

    def body(k, carry):
        slot_x(k)
        slot_y(k)
        return carry

    lax.fori_loop(0, NCHUNK, body, 0)

    pltpu.make_async_copy(rows_b, y_out.at[pl.ds(base, CHUNK)], wsem_b).wait()


def _mm_t(a, b):
    return lax.dot_general(a, b, (((1,), (1,)), ((), ())),
                           preferred_element_type=jnp.float32)


_BN = 2048


def _conv_body(ps, pc, nf, wl, bl, wr, out):
    sums = ps[...]
    pcv = pc[...]
    cnt2 = pcv[0]
    for t in range(1, NS):
        cnt2 = cnt2 + pcv[t]
    inv2 = 1.0 / jnp.maximum(cnt2, 1.0)
    invb = jnp.broadcast_to(inv2[:, None, :], (_BN // 128, 128, 128))
    invb = invb.reshape(_BN, 128)
    rowm = lax.broadcasted_iota(jnp.int32, (_BN, 128), 0) & 127
    lane = lax.broadcasted_iota(jnp.int32, (_BN, 128), 1)
    invcol = jnp.sum(jnp.where(lane == rowm, invb, 0.0), axis=1,
                     keepdims=True)
    mean = sums * invcol
    r = _mm_t(mean, wl[...]) + _mm_t(nf[...], wr[...]) + bl[...]
    out[...] = jnp.maximum(r, 0.0)


def _tc_conv(psums, pcnts, nf, wl, bl, wr):
    bn = _BN
    return pl.pallas_call(
        _conv_body,
        grid=(pl.cdiv(NP2, bn),),
        in_specs=[
            pl.BlockSpec((bn, D), lambda i: (i, 0)),
            pl.BlockSpec((NS, bn // 128, D), lambda i: (0, i, 0)),
            pl.BlockSpec((bn, D), lambda i: (i, 0)),
            pl.BlockSpec((H, D), lambda i: (0, 0)),
            pl.BlockSpec((1, H), lambda i: (0, 0)),
            pl.BlockSpec((H, D), lambda i: (0, 0)),
        ],
        out_specs=pl.BlockSpec((bn, H), lambda i: (i, 0)),
        out_shape=jax.ShapeDtypeStruct((NP2, H), jnp.float32),
    )(psums, pcnts, nf, wl, bl, wr)


def _mlp_body(x, y, wa, wb, b1, w2, b2, out):
    xv = x[...]
    yv = y[...]
    h = _mm_t(xv * yv, wa[...]) + _mm_t(xv - yv, wb[...]) + b1[...]
    h = jnp.maximum(h, 0.0)
    t = _mm_t(w2[...], h) + b2[0, 0]
    out[...] = (1.0 / (1.0 + jnp.exp(-t))).reshape(t.shape[1])


def _tc_mlp(x, y, wa, wb, b1, w2, b2):
    be = 2048
    return pl.pallas_call(
        _mlp_body,
        grid=(EP // be,),
        in_specs=[
            pl.BlockSpec((be, D), lambda i: (i, 0)),
            pl.BlockSpec((be, D), lambda i: (i, 0)),
            pl.BlockSpec((H, H), lambda i: (0, 0)),
            pl.BlockSpec((H, H), lambda i: (0, 0)),
            pl.BlockSpec((1, H), lambda i: (0, 0)),
            pl.BlockSpec((1, H), lambda i: (0, 0)),
            pl.BlockSpec(memory_space=pltpu.SMEM),
        ],
        out_specs=pl.BlockSpec((be,), lambda i: (i,)),
        out_shape=jax.ShapeDtypeStruct((EP,), jnp.float32),
    )(x, y, wa, wb, b1, w2, b2)


def kernel(node_features, edge_index, W_l, b_l, W_r, W_fc1, b_fc1, W_fc2, b_fc2):
    src = jnp.asarray(edge_index[0], jnp.int32)
    dst = jnp.asarray(edge_index[1], jnp.int32)
    pad = EP - E
    src = jnp.concatenate([src, jnp.zeros((pad,), jnp.int32)])
    dst = jnp.concatenate([dst, jnp.full((pad,), N, jnp.int32)])
    src = src.reshape(TOTCH, CHUNK)
    dst = dst.reshape(TOTCH, CHUNK)

    psums, pcnts = _sc_scatter(node_features, src, dst)
    out = _tc_conv(psums, pcnts, node_features,
                   W_l, b_l.reshape(1, H), W_r)
    x, y = _sc_gather(out, src, dst)

    wa = W_fc1[:, :H]
    wb = W_fc1[:, H:]
    p = _tc_mlp(x, y, wa, wb, b_fc1.reshape(1, H),
                W_fc2, b_fc2.reshape(1, 1))
    return p[:E].reshape(E, 1)


# speedup vs baseline: 2.7343x; 1.0842x over previous
import functools

import jax
import jax.numpy as jnp
from jax import lax
from jax.experimental import pallas as pl
from jax.experimental.pallas import tpu as pltpu
from jax.experimental.pallas import tpu_sc as plsc

N = 10000
E = 320000
D = 128
H = 128

NC = 2
NS = 16
NW = NC * NS
CHUNK = 128
NCHUNK = 80
EP = NW * NCHUNK * CHUNK
TOTCH = NW * NCHUNK
G0 = 80
G1 = 2 * NCHUNK - G0
GMAX = max(G0, G1)
NP2 = 10016
STRIPE = 624
TAIL0 = NS * STRIPE
ZR = 48
HR = 80
NPAD = HR * 128

_sc_mesh = plsc.VectorSubcoreMesh(core_axis_name="c", subcore_axis_name="s")
_sc_params = pltpu.CompilerParams(use_tc_tiling_on_sc=False,
                                  needs_layout_passes=False)


def _fill2d(ref, rows, cols, val):
    per_row = cols // 16

    def body(t, carry):
        i = t // per_row
        j = (t % per_row) * 16
        ref[i, pl.ds(j, 16)] = jnp.full((16,), val, ref.dtype)
        return carry

    lax.fori_loop(0, rows * per_row, body, 0)


def _fill1d(ref, n, val):
    def body(t, carry):
        ref[pl.ds(t * 16, 16)] = jnp.full((16,), val, ref.dtype)
        return carry

    lax.fori_loop(0, n // 16, body, 0)


DH = D // NC
TPC = TOTCH // NS


@functools.partial(
    pl.kernel,
    out_type=(
        jax.ShapeDtypeStruct((N, D), jnp.float32),
        jax.ShapeDtypeStruct((NS, HR, D), jnp.float32),
    ),
    mesh=_sc_mesh,
    compiler_params=_sc_params,
    scratch_types=(
        pltpu.VMEM((CHUNK,), jnp.int32),
        pltpu.VMEM((CHUNK,), jnp.int32),
        pltpu.VMEM((CHUNK,), jnp.int32),
        pltpu.VMEM((CHUNK,), jnp.int32),
        pltpu.VMEM((CHUNK, DH), jnp.float32),
        pltpu.VMEM((CHUNK, DH), jnp.float32),
        pltpu.VMEM((NPAD,), jnp.float32),
        pltpu.VMEM((HR, D), jnp.float32),
        pltpu.VMEM_SHARED((N, DH), jnp.float32),
        pltpu.VMEM_SHARED((NP2, DH), jnp.float32),
        pltpu.SemaphoreType.DMA,
        pltpu.SemaphoreType.DMA,
    ),
)
def _sc_scatter(nf, srci, dsti, sums_out, cnts_out,
                src_a, dst_a, src_b, dst_b, rows_a, rows_b,
                hist_v, hist2_v, nf_sh, sums_sh, sem_a, sem_b):
    c = lax.axis_index("c")
    s = lax.axis_index("s")
    col0 = c * DH

    _fill2d(rows_a, CHUNK, DH, 0.0)
    _fill1d(hist_v, NPAD, 0.0)

    row0 = s * STRIPE

    pltpu.sync_copy(nf.at[pl.ds(row0, STRIPE), pl.ds(col0, DH)],
                    nf_sh.at[pl.ds(row0, STRIPE)])

    def zero_stripe(k, carry):
        pltpu.sync_copy(rows_a, sums_sh.at[pl.ds(row0 + k * CHUNK, CHUNK)])
        return carry

    lax.fori_loop(0, 4, zero_stripe, 0)
    pltpu.sync_copy(rows_a.at[pl.ds(0, STRIPE - 4 * CHUNK)],
                    sums_sh.at[pl.ds(row0 + 4 * CHUNK, STRIPE - 4 * CHUNK)])

    @pl.when(s == 0)
    def _stage_tail():
        pltpu.sync_copy(nf.at[pl.ds(TAIL0, N - TAIL0), pl.ds(col0, DH)],
                        nf_sh.at[pl.ds(TAIL0, N - TAIL0)])
        pltpu.sync_copy(rows_a.at[pl.ds(0, 16)], sums_sh.at[pl.ds(TAIL0, 16)])

    plsc.subcore_barrier()

    ones16 = jnp.ones((16,), jnp.float32)
    cbase = s * TPC

    pltpu.sync_copy(srci.at[cbase], src_a)
    pltpu.sync_copy(dsti.at[cbase], dst_a)
    pltpu.async_copy(nf_sh.at[src_a], rows_a, sem_a)
    pltpu.sync_copy(srci.at[cbase + 1], src_b)
    pltpu.sync_copy(dsti.at[cbase + 1], dst_b)

    bufs = ((src_a, dst_a, rows_a, sem_a), (src_b, dst_b, rows_b, sem_b))

    def consume(j, cur, nxt):
        cur_s, cur_d, cur_rows, cur_sem = cur
        nxt_s, nxt_d, nxt_rows, nxt_sem = nxt

        @pl.when(j + 1 < TPC)
        def _fire_next():
            pltpu.async_copy(nf_sh.at[nxt_s], nxt_rows, nxt_sem)

        pltpu.make_async_copy(nf.at[pl.ds(0, CHUNK), pl.ds(col0, DH)],
                              cur_rows, cur_sem).wait()
        pltpu.sync_copy(cur_rows, sums_sh.at[cur_d], add=True)

        @pl.when(c == 0)
        def _hist():
            for jj in range(CHUNK // 16):
                idx = cur_d[pl.ds(jj * 16, 16)]
                plsc.addupdate_scatter(hist_v, [idx], ones16)

        @pl.when(j + 2 < TPC)
        def _prefetch_idx():
            pltpu.sync_copy(srci.at[cbase + j + 2], cur_s)
            pltpu.sync_copy(dsti.at[cbase + j + 2], cur_d)

    def body(k, carry):
        consume(2 * k, bufs[0], bufs[1])
        consume(2 * k + 1, bufs[1], bufs[0])
        return carry

    lax.fori_loop(0, TPC // 2, body, 0)

    def repack(t, carry):
        hist2_v[t // 8, pl.ds((t % 8) * 16, 16)] = hist_v[pl.ds(t * 16, 16)]
        return carry

    lax.fori_loop(0, NPAD // 16, repack, 0)
    plsc.subcore_barrier()

    pltpu.sync_copy(sums_sh.at[pl.ds(row0, STRIPE)],
                    sums_out.at[pl.ds(row0, STRIPE), pl.ds(col0, DH)])

    @pl.when(s == 0)
    def _dump_tail():
        pltpu.sync_copy(sums_sh.at[pl.ds(TAIL0, N - TAIL0)],
                        sums_out.at[pl.ds(TAIL0, N - TAIL0), pl.ds(col0, DH)])

    @pl.when(c == 0)
    def _dump_hist():
        pltpu.sync_copy(hist2_v, cnts_out.at[s])


@functools.partial(
    pl.kernel,
    out_type=(
        jax.ShapeDtypeStruct((EP, D), jnp.float32),
        jax.ShapeDtypeStruct((EP, D), jnp.float32),
    ),
    mesh=_sc_mesh,
    compiler_params=_sc_params,
    scratch_types=(
        pltpu.VMEM((CHUNK,), jnp.int32),
        pltpu.VMEM((CHUNK,), jnp.int32),
        pltpu.VMEM((CHUNK, D), jnp.float32),
        pltpu.VMEM((CHUNK, D), jnp.float32),
        pltpu.VMEM_SHARED((NP2, D), jnp.float32),
        pltpu.SemaphoreType.DMA,
        pltpu.SemaphoreType.DMA,
        pltpu.SemaphoreType.DMA,
        pltpu.SemaphoreType.DMA,
        pltpu.SemaphoreType.DMA,
        pltpu.SemaphoreType.DMA,
    ),
)
def _sc_gather(table, srci, dsti, x_out, y_out,
               ia, ib, rows_a, rows_b, table_sh,
               gsem_a, gsem_b, wsem_a, wsem_b, isem_a, isem_b):
    c = lax.axis_index("c")
    s = lax.axis_index("s")
    wid = c * NS + s
    cbase = wid * NCHUNK
    base = cbase * CHUNK

    row0 = s * STRIPE
    pltpu.sync_copy(table.at[pl.ds(row0, STRIPE)],
                    table_sh.at[pl.ds(row0, STRIPE)])

    @pl.when(s == 0)
    def _stage_tail():
        pltpu.sync_copy(table.at[pl.ds(TAIL0, NP2 - TAIL0)],
                        table_sh.at[pl.ds(TAIL0, NP2 - TAIL0)])

    plsc.subcore_barrier()

    pltpu.sync_copy(srci.at[cbase], ia)
    pltpu.async_copy(table_sh.at[ia], rows_a, gsem_a)
    pltpu.async_copy(dsti.at[cbase], ib, isem_b)

    def slot_x(k):
        @pl.when(k >= 1)
        def _drain_prev_write():
            pltpu.make_async_copy(rows_b, y_out.at[pl.ds(base, CHUNK)],
                                  wsem_b).wait()

        pltpu.make_async_copy(dsti.at[cbase], ib, isem_b).wait()
        pltpu.async_copy(table_sh.at[ib], rows_b, gsem_b)

        pltpu.make_async_copy(table.at[pl.ds(0, CHUNK)], rows_a,
                              gsem_a).wait()

        @pl.when(k + 1 < NCHUNK)
        def _fire_idx():
            pltpu.async_copy(srci.at[cbase + k + 1], ia, isem_a)

        pltpu.async_copy(rows_a, x_out.at[pl.ds(base + k * CHUNK, CHUNK)],
                         wsem_a)

    def slot_y(k):
        pltpu.make_async_copy(rows_a, x_out.at[pl.ds(base, CHUNK)],
                              wsem_a).wait()

        @pl.when(k + 1 < NCHUNK)
        def _fire_next():
            pltpu.make_async_copy(srci.at[cbase], ia, isem_a).wait()
            pltpu.async_copy(table_sh.at[ia], rows_a, gsem_a)

        pltpu.make_async_copy(table.at[pl.ds(0, CHUNK)], rows_b,
                              gsem_b).wait()

        @pl.when(k + 1 < NCHUNK)
        def _fire_idx():
            pltpu.async_copy(dsti.at[cbase + k + 1], ib, isem_b)

        pltpu.async_copy(rows_b, y_out.at[pl.ds(base + k * CHUNK, CHUNK)],
                         wsem_b)

    def body(k, carry):
        slot_x(k)
        slot_y(k)
        return carry

    lax.fori_loop(0, NCHUNK, body, 0)

    pltpu.make_async_copy(rows_b, y_out.at[pl.ds(base, CHUNK)], wsem_b).wait()


def _mm_t(a, b):
    return lax.dot_general(a, b, (((1,), (1,)), ((), ())),
                           preferred_element_type=jnp.float32)


_BN = 2048


def _conv_body(ps, pc, nf, wl, bl, wr, out):
    sums = ps[...]
    pcv = pc[...]
    cnt2 = pcv[0]
    for t in range(1, NS):
        cnt2 = cnt2 + pcv[t]
    inv2 = 1.0 / jnp.maximum(cnt2, 1.0)
    invb = jnp.broadcast_to(inv2[:, None, :], (_BN // 128, 128, 128))
    invb = invb.reshape(_BN, 128)
    rowm = lax.broadcasted_iota(jnp.int32, (_BN, 128), 0) & 127
    lane = lax.broadcasted_iota(jnp.int32, (_BN, 128), 1)
    invcol = jnp.sum(jnp.where(lane == rowm, invb, 0.0), axis=1,
                     keepdims=True)
    mean = sums * invcol
    r = _mm_t(mean, wl[...]) + _mm_t(nf[...], wr[...]) + bl[...]
    out[...] = jnp.maximum(r, 0.0)


def _tc_conv(psums, pcnts, nf, wl, bl, wr):
    bn = _BN
    return pl.pallas_call(
        _conv_body,
        grid=(pl.cdiv(NP2, bn),),
        in_specs=[
            pl.BlockSpec((bn, D), lambda i: (i, 0)),
            pl.BlockSpec((NS, bn // 128, D), lambda i: (0, i, 0)),
            pl.BlockSpec((bn, D), lambda i: (i, 0)),
            pl.BlockSpec((H, D), lambda i: (0, 0)),
            pl.BlockSpec((1, H), lambda i: (0, 0)),
            pl.BlockSpec((H, D), lambda i: (0, 0)),
        ],
        out_specs=pl.BlockSpec((bn, H), lambda i: (i, 0)),
        out_shape=jax.ShapeDtypeStruct((NP2, H), jnp.float32),
    )(psums, pcnts, nf, wl, bl, wr)


def _mlp_body(x, y, wa, wb, b1, w2, b2, out):
    xv = x[...]
    yv = y[...]
    h = _mm_t(xv * yv, wa[...]) + _mm_t(xv - yv, wb[...]) + b1[...]
    h = jnp.maximum(h, 0.0)
    t = _mm_t(w2[...], h) + b2[0, 0]
    out[...] = (1.0 / (1.0 + jnp.exp(-t))).reshape(t.shape[1])


def _tc_mlp(x, y, wa, wb, b1, w2, b2):
    be = 4096
    return pl.pallas_call(
        _mlp_body,
        grid=(EP // be,),
        in_specs=[
            pl.BlockSpec((be, D), lambda i: (i, 0)),
            pl.BlockSpec((be, D), lambda i: (i, 0)),
            pl.BlockSpec((H, H), lambda i: (0, 0)),
            pl.BlockSpec((H, H), lambda i: (0, 0)),
            pl.BlockSpec((1, H), lambda i: (0, 0)),
            pl.BlockSpec((1, H), lambda i: (0, 0)),
            pl.BlockSpec(memory_space=pltpu.SMEM),
        ],
        out_specs=pl.BlockSpec((be,), lambda i: (i,)),
        out_shape=jax.ShapeDtypeStruct((EP,), jnp.float32),
    )(x, y, wa, wb, b1, w2, b2)


def kernel(node_features, edge_index, W_l, b_l, W_r, W_fc1, b_fc1, W_fc2, b_fc2):
    src = jnp.asarray(edge_index[0], jnp.int32)
    dst = jnp.asarray(edge_index[1], jnp.int32)
    pad = EP - E
    src = jnp.concatenate([src, jnp.zeros((pad,), jnp.int32)])
    dst = jnp.concatenate([dst, jnp.full((pad,), N, jnp.int32)])
    src = src.reshape(TOTCH, CHUNK)
    dst = dst.reshape(TOTCH, CHUNK)

    psums, pcnts = _sc_scatter(node_features, src, dst)
    out = _tc_conv(psums, pcnts, node_features,
                   W_l, b_l.reshape(1, H), W_r)
    x, y = _sc_gather(out, src, dst)

    wa = W_fc1[:, :H]
    wb = W_fc1[:, H:]
    p = _tc_mlp(x, y, wa, wb, b_fc1.reshape(1, H),
                W_fc2, b_fc2.reshape(1, 1))
    return p[:E].reshape(E, 1)


# speedup vs baseline: 2.8690x; 1.0492x over previous
import functools

import jax
import jax.numpy as jnp
from jax import lax
from jax.experimental import pallas as pl
from jax.experimental.pallas import tpu as pltpu
from jax.experimental.pallas import tpu_sc as plsc

N = 10000
E = 320000
D = 128
H = 128

NC = 2
NS = 16
NW = NC * NS
CHUNK = 128
NCHUNK = 80
EP = NW * NCHUNK * CHUNK
TOTCH = NW * NCHUNK
G0 = 80
G1 = 2 * NCHUNK - G0
GMAX = max(G0, G1)
NP2 = 10016
STRIPE = 624
TAIL0 = NS * STRIPE
ZR = 48
HR = 80
NPAD = HR * 128

_sc_mesh = plsc.VectorSubcoreMesh(core_axis_name="c", subcore_axis_name="s")
_sc_params = pltpu.CompilerParams(use_tc_tiling_on_sc=False,
                                  needs_layout_passes=False)


def _fill2d(ref, rows, cols, val):
    per_row = cols // 16

    def body(t, carry):
        i = t // per_row
        j = (t % per_row) * 16
        ref[i, pl.ds(j, 16)] = jnp.full((16,), val, ref.dtype)
        return carry

    lax.fori_loop(0, rows * per_row, body, 0)


def _fill1d(ref, n, val):
    def body(t, carry):
        ref[pl.ds(t * 16, 16)] = jnp.full((16,), val, ref.dtype)
        return carry

    lax.fori_loop(0, n // 16, body, 0)


DH = D // NC
TPC = TOTCH // NS


@functools.partial(
    pl.kernel,
    out_type=(
        jax.ShapeDtypeStruct((N, D), jnp.float32),
        jax.ShapeDtypeStruct((NS, HR, D), jnp.float32),
    ),
    mesh=_sc_mesh,
    compiler_params=_sc_params,
    scratch_types=(
        pltpu.VMEM((CHUNK,), jnp.int32),
        pltpu.VMEM((CHUNK,), jnp.int32),
        pltpu.VMEM((CHUNK,), jnp.int32),
        pltpu.VMEM((CHUNK,), jnp.int32),
        pltpu.VMEM((CHUNK, DH), jnp.float32),
        pltpu.VMEM((CHUNK, DH), jnp.float32),
        pltpu.VMEM((NPAD,), jnp.float32),
        pltpu.VMEM((HR, D), jnp.float32),
        pltpu.VMEM_SHARED((N, DH), jnp.float32),
        pltpu.VMEM_SHARED((NP2, DH), jnp.float32),
        pltpu.SemaphoreType.DMA,
        pltpu.SemaphoreType.DMA,
    ),
)
def _sc_scatter(nf, srci, dsti, sums_out, cnts_out,
                src_a, dst_a, src_b, dst_b, rows_a, rows_b,
                hist_v, hist2_v, nf_sh, sums_sh, sem_a, sem_b):
    c = lax.axis_index("c")
    s = lax.axis_index("s")
    col0 = c * DH

    _fill2d(rows_a, CHUNK, DH, 0.0)
    _fill1d(hist_v, NPAD, 0.0)

    row0 = s * STRIPE

    pltpu.sync_copy(nf.at[pl.ds(row0, STRIPE), pl.ds(col0, DH)],
                    nf_sh.at[pl.ds(row0, STRIPE)])

    def zero_stripe(k, carry):
        pltpu.sync_copy(rows_a, sums_sh.at[pl.ds(row0 + k * CHUNK, CHUNK)])
        return carry

    lax.fori_loop(0, 4, zero_stripe, 0)
    pltpu.sync_copy(rows_a.at[pl.ds(0, STRIPE - 4 * CHUNK)],
                    sums_sh.at[pl.ds(row0 + 4 * CHUNK, STRIPE - 4 * CHUNK)])

    @pl.when(s == 0)
    def _stage_tail():
        pltpu.sync_copy(nf.at[pl.ds(TAIL0, N - TAIL0), pl.ds(col0, DH)],
                        nf_sh.at[pl.ds(TAIL0, N - TAIL0)])
        pltpu.sync_copy(rows_a.at[pl.ds(0, 16)], sums_sh.at[pl.ds(TAIL0, 16)])

    plsc.subcore_barrier()

    ones16 = jnp.ones((16,), jnp.float32)
    cbase = s * TPC

    pltpu.sync_copy(srci.at[cbase], src_a)
    pltpu.sync_copy(dsti.at[cbase], dst_a)
    pltpu.async_copy(nf_sh.at[src_a], rows_a, sem_a)
    pltpu.sync_copy(srci.at[cbase + 1], src_b)
    pltpu.sync_copy(dsti.at[cbase + 1], dst_b)

    bufs = ((src_a, dst_a, rows_a, sem_a), (src_b, dst_b, rows_b, sem_b))

    def consume(j, cur, nxt):
        cur_s, cur_d, cur_rows, cur_sem = cur
        nxt_s, nxt_d, nxt_rows, nxt_sem = nxt

        @pl.when(j + 1 < TPC)
        def _fire_next():
            pltpu.async_copy(nf_sh.at[nxt_s], nxt_rows, nxt_sem)

        pltpu.make_async_copy(nf.at[pl.ds(0, CHUNK), pl.ds(col0, DH)],
                              cur_rows, cur_sem).wait()
        pltpu.sync_copy(cur_rows, sums_sh.at[cur_d], add=True)

        @pl.when(c == 0)
        def _hist():
            for jj in range(CHUNK // 16):
                idx = cur_d[pl.ds(jj * 16, 16)]
                plsc.addupdate_scatter(hist_v, [idx], ones16)

        @pl.when(j + 2 < TPC)
        def _prefetch_idx():
            pltpu.sync_copy(srci.at[cbase + j + 2], cur_s)
            pltpu.sync_copy(dsti.at[cbase + j + 2], cur_d)

    def body(k, carry):
        consume(2 * k, bufs[0], bufs[1])
        consume(2 * k + 1, bufs[1], bufs[0])
        return carry

    lax.fori_loop(0, TPC // 2, body, 0)

    def repack(t, carry):
        hist2_v[t // 8, pl.ds((t % 8) * 16, 16)] = hist_v[pl.ds(t * 16, 16)]
        return carry

    lax.fori_loop(0, NPAD // 16, repack, 0)
    plsc.subcore_barrier()

    pltpu.sync_copy(sums_sh.at[pl.ds(row0, STRIPE)],
                    sums_out.at[pl.ds(row0, STRIPE), pl.ds(col0, DH)])

    @pl.when(s == 0)
    def _dump_tail():
        pltpu.sync_copy(sums_sh.at[pl.ds(TAIL0, N - TAIL0)],
                        sums_out.at[pl.ds(TAIL0, N - TAIL0), pl.ds(col0, DH)])

    @pl.when(c == 0)
    def _dump_hist():
        pltpu.sync_copy(hist2_v, cnts_out.at[s])


@functools.partial(
    pl.kernel,
    out_type=(
        jax.ShapeDtypeStruct((EP, D), jnp.float32),
        jax.ShapeDtypeStruct((EP, D), jnp.float32),
    ),
    mesh=_sc_mesh,
    compiler_params=_sc_params,
    scratch_types=(
        pltpu.VMEM((CHUNK,), jnp.int32),
        pltpu.VMEM((CHUNK,), jnp.int32),
        pltpu.VMEM((CHUNK, D), jnp.float32),
        pltpu.VMEM((CHUNK, D), jnp.float32),
        pltpu.VMEM_SHARED((NP2, D), jnp.float32),
        pltpu.SemaphoreType.DMA,
        pltpu.SemaphoreType.DMA,
        pltpu.SemaphoreType.DMA,
        pltpu.SemaphoreType.DMA,
        pltpu.SemaphoreType.DMA,
        pltpu.SemaphoreType.DMA,
    ),
)
def _sc_gather(table, srci, dsti, x_out, y_out,
               ia, ib, rows_a, rows_b, table_sh,
               gsem_a, gsem_b, wsem_a, wsem_b, isem_a, isem_b):
    c = lax.axis_index("c")
    s = lax.axis_index("s")
    wid = c * NS + s
    cbase = wid * NCHUNK
    base = cbase * CHUNK

    row0 = s * STRIPE
    pltpu.sync_copy(table.at[pl.ds(row0, STRIPE)],
                    table_sh.at[pl.ds(row0, STRIPE)])

    @pl.when(s == 0)
    def _stage_tail():
        pltpu.sync_copy(table.at[pl.ds(TAIL0, NP2 - TAIL0)],
                        table_sh.at[pl.ds(TAIL0, NP2 - TAIL0)])

    plsc.subcore_barrier()

    pltpu.sync_copy(srci.at[cbase], ia)
    pltpu.async_copy(table_sh.at[ia], rows_a, gsem_a)
    pltpu.async_copy(dsti.at[cbase], ib, isem_b)

    def slot_x(k):
        @pl.when(k >= 1)
        def _drain_prev_write():
            pltpu.make_async_copy(rows_b, y_out.at[pl.ds(base, CHUNK)],
                                  wsem_b).wait()

        pltpu.make_async_copy(dsti.at[cbase], ib, isem_b).wait()
        pltpu.async_copy(table_sh.at[ib], rows_b, gsem_b)

        pltpu.make_async_copy(table.at[pl.ds(0, CHUNK)], rows_a,
                              gsem_a).wait()

        @pl.when(k + 1 < NCHUNK)
        def _fire_idx():
            pltpu.async_copy(srci.at[cbase + k + 1], ia, isem_a)

        pltpu.async_copy(rows_a, x_out.at[pl.ds(base + k * CHUNK, CHUNK)],
                         wsem_a)

    def slot_y(k):
        pltpu.make_async_copy(rows_a, x_out.at[pl.ds(base, CHUNK)],
                              wsem_a).wait()

        @pl.when(k + 1 < NCHUNK)
        def _fire_next():
            pltpu.make_async_copy(srci.at[cbase], ia, isem_a).wait()
            pltpu.async_copy(table_sh.at[ia], rows_a, gsem_a)

        pltpu.make_async_copy(table.at[pl.ds(0, CHUNK)], rows_b,
                              gsem_b).wait()

        @pl.when(k + 1 < NCHUNK)
        def _fire_idx():
            pltpu.async_copy(dsti.at[cbase + k + 1], ib, isem_b)

        pltpu.async_copy(rows_b, y_out.at[pl.ds(base + k * CHUNK, CHUNK)],
                         wsem_b)

    def body(k, carry):
        slot_x(k)
        slot_y(k)
        return carry

    lax.fori_loop(0, NCHUNK, body, 0)

    pltpu.make_async_copy(rows_b, y_out.at[pl.ds(base, CHUNK)], wsem_b).wait()


def _mm_t(a, b):
    return lax.dot_general(a, b, (((1,), (1,)), ((), ())),
                           preferred_element_type=jnp.float32)


_BN = 2048


def _conv_body(ps, pc, nf, wl, bl, wr, out):
    sums = ps[...]
    pcv = pc[...]
    cnt2 = pcv[0]
    for t in range(1, NS):
        cnt2 = cnt2 + pcv[t]
    inv2 = 1.0 / jnp.maximum(cnt2, 1.0)
    invb = jnp.broadcast_to(inv2[:, None, :], (_BN // 128, 128, 128))
    invb = invb.reshape(_BN, 128)
    rowm = lax.broadcasted_iota(jnp.int32, (_BN, 128), 0) & 127
    lane = lax.broadcasted_iota(jnp.int32, (_BN, 128), 1)
    invcol = jnp.sum(jnp.where(lane == rowm, invb, 0.0), axis=1,
                     keepdims=True)
    mean = sums * invcol
    r = _mm_t(mean, wl[...]) + _mm_t(nf[...], wr[...]) + bl[...]
    out[...] = jnp.maximum(r, 0.0)


def _tc_conv(psums, pcnts, nf, wl, bl, wr):
    bn = _BN
    return pl.pallas_call(
        _conv_body,
        grid=(pl.cdiv(NP2, bn),),
        in_specs=[
            pl.BlockSpec((bn, D), lambda i: (i, 0)),
            pl.BlockSpec((NS, bn // 128, D), lambda i: (0, i, 0)),
            pl.BlockSpec((bn, D), lambda i: (i, 0)),
            pl.BlockSpec((H, D), lambda i: (0, 0)),
            pl.BlockSpec((1, H), lambda i: (0, 0)),
            pl.BlockSpec((H, D), lambda i: (0, 0)),
        ],
        out_specs=pl.BlockSpec((bn, H), lambda i: (i, 0)),
        out_shape=jax.ShapeDtypeStruct((NP2, H), jnp.float32),
    )(psums, pcnts, nf, wl, bl, wr)


def _mlp_body(x, y, wa, wb, b1, w2, b2, out):
    xv = x[...]
    yv = y[...]
    h = _mm_t(xv * yv, wa[...]) + _mm_t(xv - yv, wb[...]) + b1[...]
    h = jnp.maximum(h, 0.0)
    t = _mm_t(w2[...], h) + b2[0, 0]
    out[...] = (1.0 / (1.0 + jnp.exp(-t))).reshape(t.shape[1])


def _tc_mlp(x, y, wa, wb, b1, w2, b2):
    be = 8192
    return pl.pallas_call(
        _mlp_body,
        grid=(EP // be,),
        in_specs=[
            pl.BlockSpec((be, D), lambda i: (i, 0)),
            pl.BlockSpec((be, D), lambda i: (i, 0)),
            pl.BlockSpec((H, H), lambda i: (0, 0)),
            pl.BlockSpec((H, H), lambda i: (0, 0)),
            pl.BlockSpec((1, H), lambda i: (0, 0)),
            pl.BlockSpec((1, H), lambda i: (0, 0)),
            pl.BlockSpec(memory_space=pltpu.SMEM),
        ],
        out_specs=pl.BlockSpec((be,), lambda i: (i,)),
        out_shape=jax.ShapeDtypeStruct((EP,), jnp.float32),
    )(x, y, wa, wb, b1, w2, b2)


def kernel(node_features, edge_index, W_l, b_l, W_r, W_fc1, b_fc1, W_fc2, b_fc2):
    src = jnp.asarray(edge_index[0], jnp.int32)
    dst = jnp.asarray(edge_index[1], jnp.int32)
    pad = EP - E
    src = jnp.concatenate([src, jnp.zeros((pad,), jnp.int32)])
    dst = jnp.concatenate([dst, jnp.full((pad,), N, jnp.int32)])
    src = src.reshape(TOTCH, CHUNK)
    dst = dst.reshape(TOTCH, CHUNK)

    psums, pcnts = _sc_scatter(node_features, src, dst)
    out = _tc_conv(psums, pcnts, node_features,
                   W_l, b_l.reshape(1, H), W_r)
    x, y = _sc_gather(out, src, dst)

    wa = W_fc1[:, :H]
    wb = W_fc1[:, H:]
    p = _tc_mlp(x, y, wa, wb, b_fc1.reshape(1, H),
                W_fc2, b_fc2.reshape(1, 1))
    return p[:E].reshape(E, 1)


# speedup vs baseline: 2.9081x; 1.0137x over previous
import functools

import jax
import jax.numpy as jnp
from jax import lax
from jax.experimental import pallas as pl
from jax.experimental.pallas import tpu as pltpu
from jax.experimental.pallas import tpu_sc as plsc

N = 10000
E = 320000
D = 128
H = 128

NC = 2
NS = 16
NW = NC * NS
CHUNK = 128
NCHUNK = 80
EP = NW * NCHUNK * CHUNK
TOTCH = NW * NCHUNK
G0 = 80
G1 = 2 * NCHUNK - G0
GMAX = max(G0, G1)
NP2 = 10016
STRIPE = 624
TAIL0 = NS * STRIPE
ZR = 48
HR = 80
NPAD = HR * 128

_sc_mesh = plsc.VectorSubcoreMesh(core_axis_name="c", subcore_axis_name="s")
_sc_params = pltpu.CompilerParams(use_tc_tiling_on_sc=False,
                                  needs_layout_passes=False)


def _fill2d(ref, rows, cols, val):
    per_row = cols // 16

    def body(t, carry):
        i = t // per_row
        j = (t % per_row) * 16
        ref[i, pl.ds(j, 16)] = jnp.full((16,), val, ref.dtype)
        return carry

    lax.fori_loop(0, rows * per_row, body, 0)


def _fill1d(ref, n, val):
    def body(t, carry):
        ref[pl.ds(t * 16, 16)] = jnp.full((16,), val, ref.dtype)
        return carry

    lax.fori_loop(0, n // 16, body, 0)


DH = D // NC
TPC = TOTCH // NS


@functools.partial(
    pl.kernel,
    out_type=(
        jax.ShapeDtypeStruct((N, D), jnp.float32),
        jax.ShapeDtypeStruct((NS, HR, D), jnp.float32),
    ),
    mesh=_sc_mesh,
    compiler_params=_sc_params,
    scratch_types=(
        pltpu.VMEM((CHUNK,), jnp.int32),
        pltpu.VMEM((CHUNK,), jnp.int32),
        pltpu.VMEM((CHUNK,), jnp.int32),
        pltpu.VMEM((CHUNK,), jnp.int32),
        pltpu.VMEM((CHUNK, DH), jnp.float32),
        pltpu.VMEM((CHUNK, DH), jnp.float32),
        pltpu.VMEM((NPAD,), jnp.float32),
        pltpu.VMEM((HR, D), jnp.float32),
        pltpu.VMEM_SHARED((N, DH), jnp.float32),
        pltpu.VMEM_SHARED((NP2, DH), jnp.float32),
        pltpu.SemaphoreType.DMA,
        pltpu.SemaphoreType.DMA,
    ),
)
def _sc_scatter(nf, srci, dsti, sums_out, cnts_out,
                src_a, dst_a, src_b, dst_b, rows_a, rows_b,
                hist_v, hist2_v, nf_sh, sums_sh, sem_a, sem_b):
    c = lax.axis_index("c")
    s = lax.axis_index("s")
    col0 = c * DH

    _fill2d(rows_a, CHUNK, DH, 0.0)
    _fill1d(hist_v, NPAD, 0.0)

    row0 = s * STRIPE

    pltpu.sync_copy(nf.at[pl.ds(row0, STRIPE), pl.ds(col0, DH)],
                    nf_sh.at[pl.ds(row0, STRIPE)])

    def zero_stripe(k, carry):
        pltpu.sync_copy(rows_a, sums_sh.at[pl.ds(row0 + k * CHUNK, CHUNK)])
        return carry

    lax.fori_loop(0, 4, zero_stripe, 0)
    pltpu.sync_copy(rows_a.at[pl.ds(0, STRIPE - 4 * CHUNK)],
                    sums_sh.at[pl.ds(row0 + 4 * CHUNK, STRIPE - 4 * CHUNK)])

    @pl.when(s == 0)
    def _stage_tail():
        pltpu.sync_copy(nf.at[pl.ds(TAIL0, N - TAIL0), pl.ds(col0, DH)],
                        nf_sh.at[pl.ds(TAIL0, N - TAIL0)])
        pltpu.sync_copy(rows_a.at[pl.ds(0, 16)], sums_sh.at[pl.ds(TAIL0, 16)])

    plsc.subcore_barrier()

    ones16 = jnp.ones((16,), jnp.float32)
    cbase = s * TPC

    pltpu.sync_copy(srci.at[cbase], src_a)
    pltpu.sync_copy(dsti.at[cbase], dst_a)
    pltpu.async_copy(nf_sh.at[src_a], rows_a, sem_a)
    pltpu.sync_copy(srci.at[cbase + 1], src_b)
    pltpu.sync_copy(dsti.at[cbase + 1], dst_b)

    bufs = ((src_a, dst_a, rows_a, sem_a), (src_b, dst_b, rows_b, sem_b))

    def consume(j, cur, nxt):
        cur_s, cur_d, cur_rows, cur_sem = cur
        nxt_s, nxt_d, nxt_rows, nxt_sem = nxt

        @pl.when(j + 1 < TPC)
        def _fire_next():
            pltpu.async_copy(nf_sh.at[nxt_s], nxt_rows, nxt_sem)

        pltpu.make_async_copy(nf.at[pl.ds(0, CHUNK), pl.ds(col0, DH)],
                              cur_rows, cur_sem).wait()
        pltpu.sync_copy(cur_rows, sums_sh.at[cur_d], add=True)

        @pl.when(c == 0)
        def _hist():
            for jj in range(CHUNK // 16):
                idx = cur_d[pl.ds(jj * 16, 16)]
                plsc.addupdate_scatter(hist_v, [idx], ones16)

        @pl.when(j + 2 < TPC)
        def _prefetch_idx():
            pltpu.sync_copy(srci.at[cbase + j + 2], cur_s)
            pltpu.sync_copy(dsti.at[cbase + j + 2], cur_d)

    def body(k, carry):
        consume(2 * k, bufs[0], bufs[1])
        consume(2 * k + 1, bufs[1], bufs[0])
        return carry

    lax.fori_loop(0, TPC // 2, body, 0)

    def repack(t, carry):
        hist2_v[t // 8, pl.ds((t % 8) * 16, 16)] = hist_v[pl.ds(t * 16, 16)]
        return carry

    lax.fori_loop(0, NPAD // 16, repack, 0)
    plsc.subcore_barrier()

    pltpu.sync_copy(sums_sh.at[pl.ds(row0, STRIPE)],
                    sums_out.at[pl.ds(row0, STRIPE), pl.ds(col0, DH)])

    @pl.when(s == 0)
    def _dump_tail():
        pltpu.sync_copy(sums_sh.at[pl.ds(TAIL0, N - TAIL0)],
                        sums_out.at[pl.ds(TAIL0, N - TAIL0), pl.ds(col0, DH)])

    @pl.when(c == 0)
    def _dump_hist():
        pltpu.sync_copy(hist2_v, cnts_out.at[s])


@functools.partial(
    pl.kernel,
    out_type=(
        jax.ShapeDtypeStruct((EP, D), jnp.float32),
        jax.ShapeDtypeStruct((EP, D), jnp.float32),
    ),
    mesh=_sc_mesh,
    compiler_params=_sc_params,
    scratch_types=(
        pltpu.VMEM((CHUNK,), jnp.int32),
        pltpu.VMEM((CHUNK,), jnp.int32),
        pltpu.VMEM((CHUNK, D), jnp.float32),
        pltpu.VMEM((CHUNK, D), jnp.float32),
        pltpu.VMEM_SHARED((NP2, D), jnp.float32),
        pltpu.SemaphoreType.DMA,
        pltpu.SemaphoreType.DMA,
        pltpu.SemaphoreType.DMA,
        pltpu.SemaphoreType.DMA,
        pltpu.SemaphoreType.DMA,
        pltpu.SemaphoreType.DMA,
    ),
)
def _sc_gather(table, srci, dsti, x_out, y_out,
               ia, ib, rows_a, rows_b, table_sh,
               gsem_a, gsem_b, wsem_a, wsem_b, isem_a, isem_b):
    c = lax.axis_index("c")
    s = lax.axis_index("s")
    wid = c * NS + s
    cbase = wid * NCHUNK
    base = cbase * CHUNK

    row0 = s * STRIPE
    pltpu.sync_copy(table.at[pl.ds(row0, STRIPE)],
                    table_sh.at[pl.ds(row0, STRIPE)])

    @pl.when(s == 0)
    def _stage_tail():
        pltpu.sync_copy(table.at[pl.ds(TAIL0, NP2 - TAIL0)],
                        table_sh.at[pl.ds(TAIL0, NP2 - TAIL0)])

    plsc.subcore_barrier()

    pltpu.sync_copy(srci.at[cbase], ia)
    pltpu.async_copy(table_sh.at[ia], rows_a, gsem_a)
    pltpu.async_copy(dsti.at[cbase], ib, isem_b)

    def slot_x(k):
        @pl.when(k >= 1)
        def _drain_prev_write():
            pltpu.make_async_copy(rows_b, y_out.at[pl.ds(base, CHUNK)],
                                  wsem_b).wait()

        pltpu.make_async_copy(dsti.at[cbase], ib, isem_b).wait()
        pltpu.async_copy(table_sh.at[ib], rows_b, gsem_b)

        pltpu.make_async_copy(table.at[pl.ds(0, CHUNK)], rows_a,
                              gsem_a).wait()

        @pl.when(k + 1 < NCHUNK)
        def _fire_idx():
            pltpu.async_copy(srci.at[cbase + k + 1], ia, isem_a)

        pltpu.async_copy(rows_a, x_out.at[pl.ds(base + k * CHUNK, CHUNK)],
                         wsem_a)

    def slot_y(k):
        pltpu.make_async_copy(rows_a, x_out.at[pl.ds(base, CHUNK)],
                              wsem_a).wait()

        @pl.when(k + 1 < NCHUNK)
        def _fire_next():
            pltpu.make_async_copy(srci.at[cbase], ia, isem_a).wait()
            pltpu.async_copy(table_sh.at[ia], rows_a, gsem_a)

        pltpu.make_async_copy(table.at[pl.ds(0, CHUNK)], rows_b,
                              gsem_b).wait()

        @pl.when(k + 1 < NCHUNK)
        def _fire_idx():
            pltpu.async_copy(dsti.at[cbase + k + 1], ib, isem_b)

        pltpu.async_copy(rows_b, y_out.at[pl.ds(base + k * CHUNK, CHUNK)],
                         wsem_b)

    def body(k, carry):
        slot_x(k)
        slot_y(k)
        return carry

    lax.fori_loop(0, NCHUNK, body, 0)

    pltpu.make_async_copy(rows_b, y_out.at[pl.ds(base, CHUNK)], wsem_b).wait()


def _mm_t(a, b):
    return lax.dot_general(a, b, (((1,), (1,)), ((), ())),
                           preferred_element_type=jnp.float32)


_BN = 2048


def _conv_body(ps, pc, nf, wl, bl, wr, out):
    sums = ps[...]
    pcv = pc[...]
    cnt2 = pcv[0]
    for t in range(1, NS):
        cnt2 = cnt2 + pcv[t]
    inv2 = 1.0 / jnp.maximum(cnt2, 1.0)
    invb = jnp.broadcast_to(inv2[:, None, :], (_BN // 128, 128, 128))
    invb = invb.reshape(_BN, 128)
    rowm = lax.broadcasted_iota(jnp.int32, (_BN, 128), 0) & 127
    lane = lax.broadcasted_iota(jnp.int32, (_BN, 128), 1)
    invcol = jnp.sum(jnp.where(lane == rowm, invb, 0.0), axis=1,
                     keepdims=True)
    mean = sums * invcol
    r = _mm_t(mean, wl[...]) + _mm_t(nf[...], wr[...]) + bl[...]
    out[...] = jnp.maximum(r, 0.0)


def _tc_conv(psums, pcnts, nf, wl, bl, wr):
    bn = _BN
    return pl.pallas_call(
        _conv_body,
        grid=(pl.cdiv(NP2, bn),),
        in_specs=[
            pl.BlockSpec((bn, D), lambda i: (i, 0)),
            pl.BlockSpec((NS, bn // 128, D), lambda i: (0, i, 0)),
            pl.BlockSpec((bn, D), lambda i: (i, 0)),
            pl.BlockSpec((H, D), lambda i: (0, 0)),
            pl.BlockSpec((1, H), lambda i: (0, 0)),
            pl.BlockSpec((H, D), lambda i: (0, 0)),
        ],
        out_specs=pl.BlockSpec((bn, H), lambda i: (i, 0)),
        out_shape=jax.ShapeDtypeStruct((NP2, H), jnp.float32),
    )(psums, pcnts, nf, wl, bl, wr)


def _mlp_body(x, y, wa, wb, b1, w2, b2, out):
    xv = x[...]
    yv = y[...]
    h = _mm_t(xv * yv, wa[...]) + _mm_t(xv - yv, wb[...]) + b1[...]
    h = jnp.maximum(h, 0.0)
    t = _mm_t(w2[...], h) + b2[0, 0]
    out[...] = (1.0 / (1.0 + jnp.exp(-t))).reshape(t.shape[1])


def _tc_mlp(x, y, wa, wb, b1, w2, b2):
    be = 16384
    return pl.pallas_call(
        _mlp_body,
        grid=(EP // be,),
        in_specs=[
            pl.BlockSpec((be, D), lambda i: (i, 0)),
            pl.BlockSpec((be, D), lambda i: (i, 0)),
            pl.BlockSpec((H, H), lambda i: (0, 0)),
            pl.BlockSpec((H, H), lambda i: (0, 0)),
            pl.BlockSpec((1, H), lambda i: (0, 0)),
            pl.BlockSpec((1, H), lambda i: (0, 0)),
            pl.BlockSpec(memory_space=pltpu.SMEM),
        ],
        out_specs=pl.BlockSpec((be,), lambda i: (i,)),
        out_shape=jax.ShapeDtypeStruct((EP,), jnp.float32),
    )(x, y, wa, wb, b1, w2, b2)


def kernel(node_features, edge_index, W_l, b_l, W_r, W_fc1, b_fc1, W_fc2, b_fc2):
    src = jnp.asarray(edge_index[0], jnp.int32)
    dst = jnp.asarray(edge_index[1], jnp.int32)
    pad = EP - E
    src = jnp.concatenate([src, jnp.zeros((pad,), jnp.int32)])
    dst = jnp.concatenate([dst, jnp.full((pad,), N, jnp.int32)])
    src = src.reshape(TOTCH, CHUNK)
    dst = dst.reshape(TOTCH, CHUNK)

    psums, pcnts = _sc_scatter(node_features, src, dst)
    out = _tc_conv(psums, pcnts, node_features,
                   W_l, b_l.reshape(1, H), W_r)
    x, y = _sc_gather(out, src, dst)

    wa = W_fc1[:, :H]
    wb = W_fc1[:, H:]
    p = _tc_mlp(x, y, wa, wb, b_fc1.reshape(1, H),
                W_fc2, b_fc2.reshape(1, 1))
    return p[:E].reshape(E, 1)


# speedup vs baseline: 2.9088x; 1.0002x over previous
import functools

import jax
import jax.numpy as jnp
from jax import lax
from jax.experimental import pallas as pl
from jax.experimental.pallas import tpu as pltpu
from jax.experimental.pallas import tpu_sc as plsc

N = 10000
E = 320000
D = 128
H = 128

NC = 2
NS = 16
NW = NC * NS
CHUNK = 128
NCHUNK = 80
EP = NW * NCHUNK * CHUNK
TOTCH = NW * NCHUNK
NP2 = 10016
STRIPE = 624
TAIL0 = NS * STRIPE
HR = 80
NPAD = HR * 128

_sc_mesh = plsc.VectorSubcoreMesh(core_axis_name="c", subcore_axis_name="s")
_sc_params = pltpu.CompilerParams(use_tc_tiling_on_sc=False,
                                  needs_layout_passes=False)


def _fill2d(ref, rows, cols, val):
    per_row = cols // 16

    def body(t, carry):
        i = t // per_row
        j = (t % per_row) * 16
        ref[i, pl.ds(j, 16)] = jnp.full((16,), val, ref.dtype)
        return carry

    lax.fori_loop(0, rows * per_row, body, 0)


def _fill1d(ref, n, val):
    def body(t, carry):
        ref[pl.ds(t * 16, 16)] = jnp.full((16,), val, ref.dtype)
        return carry

    lax.fori_loop(0, n // 16, body, 0)


DH = D // NC
TPC = TOTCH // NS


@functools.partial(
    pl.kernel,
    out_type=(
        jax.ShapeDtypeStruct((N, D), jnp.float32),
        jax.ShapeDtypeStruct((NS, HR, D), jnp.float32),
    ),
    mesh=_sc_mesh,
    compiler_params=_sc_params,
    scratch_types=(
        pltpu.VMEM((CHUNK,), jnp.int32),
        pltpu.VMEM((CHUNK,), jnp.int32),
        pltpu.VMEM((CHUNK,), jnp.int32),
        pltpu.VMEM((CHUNK,), jnp.int32),
        pltpu.VMEM((CHUNK, DH), jnp.float32),
        pltpu.VMEM((CHUNK, DH), jnp.float32),
        pltpu.VMEM((NPAD,), jnp.float32),
        pltpu.VMEM((HR, D), jnp.float32),
        pltpu.VMEM_SHARED((N, DH), jnp.float32),
        pltpu.VMEM_SHARED((NP2, DH), jnp.float32),
        pltpu.SemaphoreType.DMA,
        pltpu.SemaphoreType.DMA,
    ),
)
def _sc_scatter(nf, srci, dsti, sums_out, cnts_out,
                src_a, dst_a, src_b, dst_b, rows_a, rows_b,
                hist_v, hist2_v, nf_sh, sums_sh, sem_a, sem_b):
    c = lax.axis_index("c")
    s = lax.axis_index("s")
    col0 = c * DH

    _fill2d(rows_a, CHUNK, DH, 0.0)
    _fill1d(hist_v, NPAD, 0.0)

    row0 = s * STRIPE

    pltpu.sync_copy(nf.at[pl.ds(row0, STRIPE), pl.ds(col0, DH)],
                    nf_sh.at[pl.ds(row0, STRIPE)])

    def zero_stripe(k, carry):
        pltpu.sync_copy(rows_a, sums_sh.at[pl.ds(row0 + k * CHUNK, CHUNK)])
        return carry

    lax.fori_loop(0, 4, zero_stripe, 0)
    pltpu.sync_copy(rows_a.at[pl.ds(0, STRIPE - 4 * CHUNK)],
                    sums_sh.at[pl.ds(row0 + 4 * CHUNK, STRIPE - 4 * CHUNK)])

    @pl.when(s == 0)
    def _stage_tail():
        pltpu.sync_copy(nf.at[pl.ds(TAIL0, N - TAIL0), pl.ds(col0, DH)],
                        nf_sh.at[pl.ds(TAIL0, N - TAIL0)])
        pltpu.sync_copy(rows_a.at[pl.ds(0, 16)], sums_sh.at[pl.ds(TAIL0, 16)])

    plsc.subcore_barrier()

    ones16 = jnp.ones((16,), jnp.float32)
    cbase = s * TPC

    pltpu.sync_copy(srci.at[cbase], src_a)
    pltpu.sync_copy(dsti.at[cbase], dst_a)
    pltpu.async_copy(nf_sh.at[src_a], rows_a, sem_a)
    pltpu.sync_copy(srci.at[cbase + 1], src_b)
    pltpu.sync_copy(dsti.at[cbase + 1], dst_b)

    bufs = ((src_a, dst_a, rows_a, sem_a), (src_b, dst_b, rows_b, sem_b))

    def consume(j, cur, nxt):
        cur_s, cur_d, cur_rows, cur_sem = cur
        nxt_s, nxt_d, nxt_rows, nxt_sem = nxt

        @pl.when(j + 1 < TPC)
        def _fire_next():
            pltpu.async_copy(nf_sh.at[nxt_s], nxt_rows, nxt_sem)

        pltpu.make_async_copy(nf.at[pl.ds(0, CHUNK), pl.ds(col0, DH)],
                              cur_rows, cur_sem).wait()
        pltpu.sync_copy(cur_rows, sums_sh.at[cur_d], add=True)

        @pl.when(c == 0)
        def _hist():
            for jj in range(CHUNK // 16):
                idx = cur_d[pl.ds(jj * 16, 16)]
                plsc.addupdate_scatter(hist_v, [idx], ones16)

        @pl.when(j + 2 < TPC)
        def _prefetch_idx():
            pltpu.sync_copy(srci.at[cbase + j + 2], cur_s)
            pltpu.sync_copy(dsti.at[cbase + j + 2], cur_d)

    def body(k, carry):
        consume(2 * k, bufs[0], bufs[1])
        consume(2 * k + 1, bufs[1], bufs[0])
        return carry

    lax.fori_loop(0, TPC // 2, body, 0)

    def repack(t, carry):
        hist2_v[t // 8, pl.ds((t % 8) * 16, 16)] = hist_v[pl.ds(t * 16, 16)]
        return carry

    lax.fori_loop(0, NPAD // 16, repack, 0)
    plsc.subcore_barrier()

    pltpu.sync_copy(sums_sh.at[pl.ds(row0, STRIPE)],
                    sums_out.at[pl.ds(row0, STRIPE), pl.ds(col0, DH)])

    @pl.when(s == 0)
    def _dump_tail():
        pltpu.sync_copy(sums_sh.at[pl.ds(TAIL0, N - TAIL0)],
                        sums_out.at[pl.ds(TAIL0, N - TAIL0), pl.ds(col0, DH)])

    @pl.when(c == 0)
    def _dump_hist():
        pltpu.sync_copy(hist2_v, cnts_out.at[s])


@functools.partial(
    pl.kernel,
    out_type=(
        jax.ShapeDtypeStruct((EP, D), jnp.float32),
        jax.ShapeDtypeStruct((EP, D), jnp.float32),
    ),
    mesh=_sc_mesh,
    compiler_params=_sc_params,
    scratch_types=(
        pltpu.VMEM((CHUNK,), jnp.int32),
        pltpu.VMEM((CHUNK,), jnp.int32),
        pltpu.VMEM((CHUNK, D), jnp.float32),
        pltpu.VMEM((CHUNK, D), jnp.float32),
        pltpu.VMEM_SHARED((NP2, D), jnp.float32),
        pltpu.SemaphoreType.DMA,
        pltpu.SemaphoreType.DMA,
        pltpu.SemaphoreType.DMA,
        pltpu.SemaphoreType.DMA,
        pltpu.SemaphoreType.DMA,
        pltpu.SemaphoreType.DMA,
    ),
)
def _sc_gather(table, srci, dsti, x_out, y_out,
               ia, ib, rows_a, rows_b, table_sh,
               gsem_a, gsem_b, wsem_a, wsem_b, isem_a, isem_b):
    c = lax.axis_index("c")
    s = lax.axis_index("s")
    wid = c * NS + s
    cbase = wid * NCHUNK
    base = cbase * CHUNK

    row0 = s * STRIPE
    pltpu.sync_copy(table.at[pl.ds(row0, STRIPE)],
                    table_sh.at[pl.ds(row0, STRIPE)])

    @pl.when(s == 0)
    def _stage_tail():
        pltpu.sync_copy(table.at[pl.ds(TAIL0, NP2 - TAIL0)],
                        table_sh.at[pl.ds(TAIL0, NP2 - TAIL0)])

    plsc.subcore_barrier()

    pltpu.sync_copy(srci.at[cbase], ia)
    pltpu.async_copy(table_sh.at[ia], rows_a, gsem_a)
    pltpu.async_copy(dsti.at[cbase], ib, isem_b)

    def slot_x(k):
        @pl.when(k >= 1)
        def _drain_prev_write():
            pltpu.make_async_copy(rows_b, y_out.at[pl.ds(base, CHUNK)],
                                  wsem_b).wait()

        pltpu.make_async_copy(dsti.at[cbase], ib, isem_b).wait()
        pltpu.async_copy(table_sh.at[ib], rows_b, gsem_b)

        pltpu.make_async_copy(table.at[pl.ds(0, CHUNK)], rows_a,
                              gsem_a).wait()

        @pl.when(k + 1 < NCHUNK)
        def _fire_idx():
            pltpu.async_copy(srci.at[cbase + k + 1], ia, isem_a)

        pltpu.async_copy(rows_a, x_out.at[pl.ds(base + k * CHUNK, CHUNK)],
                         wsem_a)

    def slot_y(k):
        pltpu.make_async_copy(rows_a, x_out.at[pl.ds(base, CHUNK)],
                              wsem_a).wait()

        @pl.when(k + 1 < NCHUNK)
        def _fire_next():
            pltpu.make_async_copy(srci.at[cbase], ia, isem_a).wait()
            pltpu.async_copy(table_sh.at[ia], rows_a, gsem_a)

        pltpu.make_async_copy(table.at[pl.ds(0, CHUNK)], rows_b,
                              gsem_b).wait()

        @pl.when(k + 1 < NCHUNK)
        def _fire_idx():
            pltpu.async_copy(dsti.at[cbase + k + 1], ib, isem_b)

        pltpu.async_copy(rows_b, y_out.at[pl.ds(base + k * CHUNK, CHUNK)],
                         wsem_b)

    def body(k, carry):
        slot_x(k)
        slot_y(k)
        return carry

    lax.fori_loop(0, NCHUNK, body, 0)

    pltpu.make_async_copy(rows_b, y_out.at[pl.ds(base, CHUNK)], wsem_b).wait()


def _mm_t(a, b):
    return lax.dot_general(a, b, (((1,), (1,)), ((), ())),
                           preferred_element_type=jnp.float32)


_BN = 2048


def _conv_body(ps, pc, nf, wl, bl, wr, out):
    sums = ps[...]
    pcv = pc[...]
    cnt2 = pcv[0]
    for t in range(1, NS):
        cnt2 = cnt2 + pcv[t]
    inv2 = 1.0 / jnp.maximum(cnt2, 1.0)
    invb = jnp.broadcast_to(inv2[:, None, :], (_BN // 128, 128, 128))
    invb = invb.reshape(_BN, 128)
    rowm = lax.broadcasted_iota(jnp.int32, (_BN, 128), 0) & 127
    lane = lax.broadcasted_iota(jnp.int32, (_BN, 128), 1)
    invcol = jnp.sum(jnp.where(lane == rowm, invb, 0.0), axis=1,
                     keepdims=True)
    mean = sums * invcol
    r = _mm_t(mean, wl[...]) + _mm_t(nf[...], wr[...]) + bl[...]
    out[...] = jnp.maximum(r, 0.0)


def _tc_conv(psums, pcnts, nf, wl, bl, wr):
    bn = _BN
    return pl.pallas_call(
        _conv_body,
        grid=(pl.cdiv(NP2, bn),),
        in_specs=[
            pl.BlockSpec((bn, D), lambda i: (i, 0)),
            pl.BlockSpec((NS, bn // 128, D), lambda i: (0, i, 0)),
            pl.BlockSpec((bn, D), lambda i: (i, 0)),
            pl.BlockSpec((H, D), lambda i: (0, 0)),
            pl.BlockSpec((1, H), lambda i: (0, 0)),
            pl.BlockSpec((H, D), lambda i: (0, 0)),
        ],
        out_specs=pl.BlockSpec((bn, H), lambda i: (i, 0)),
        out_shape=jax.ShapeDtypeStruct((NP2, H), jnp.float32),
    )(psums, pcnts, nf, wl, bl, wr)


def _mlp_body(x, y, wa, wb, b1, w2, b2, out):
    xv = x[...]
    yv = y[...]
    h = _mm_t(xv * yv, wa[...]) + _mm_t(xv - yv, wb[...]) + b1[...]
    h = jnp.maximum(h, 0.0)
    t = _mm_t(w2[...], h) + b2[0, 0]
    out[...] = (1.0 / (1.0 + jnp.exp(-t))).reshape(t.shape[1])


def _tc_mlp(x, y, wa, wb, b1, w2, b2):
    be = 16384
    return pl.pallas_call(
        _mlp_body,
        grid=(EP // be,),
        in_specs=[
            pl.BlockSpec((be, D), lambda i: (i, 0)),
            pl.BlockSpec((be, D), lambda i: (i, 0)),
            pl.BlockSpec((H, H), lambda i: (0, 0)),
            pl.BlockSpec((H, H), lambda i: (0, 0)),
            pl.BlockSpec((1, H), lambda i: (0, 0)),
            pl.BlockSpec((1, H), lambda i: (0, 0)),
            pl.BlockSpec(memory_space=pltpu.SMEM),
        ],
        out_specs=pl.BlockSpec((be,), lambda i: (i,)),
        out_shape=jax.ShapeDtypeStruct((EP,), jnp.float32),
    )(x, y, wa, wb, b1, w2, b2)


def kernel(node_features, edge_index, W_l, b_l, W_r, W_fc1, b_fc1, W_fc2, b_fc2):
    src = jnp.asarray(edge_index[0], jnp.int32)
    dst = jnp.asarray(edge_index[1], jnp.int32)
    pad = EP - E
    src = jnp.concatenate([src, jnp.zeros((pad,), jnp.int32)])
    dst = jnp.concatenate([dst, jnp.full((pad,), N, jnp.int32)])
    src = src.reshape(TOTCH, CHUNK)
    dst = dst.reshape(TOTCH, CHUNK)

    psums, pcnts = _sc_scatter(node_features, src, dst)
    out = _tc_conv(psums, pcnts, node_features,
                   W_l, b_l.reshape(1, H), W_r)
    x, y = _sc_gather(out, src, dst)

    wa = W_fc1[:, :H]
    wb = W_fc1[:, H:]
    p = _tc_mlp(x, y, wa, wb, b_fc1.reshape(1, H),
                W_fc2, b_fc2.reshape(1, 1))
    return p[:E].reshape(E, 1)
